# baseline (speedup 1.0000x reference)
import functools

import jax
import jax.numpy as jnp
from jax import lax
from jax.experimental import pallas as pl
from jax.experimental.pallas import tpu as pltpu
from jax.experimental.pallas import tpu_sc as plsc

N = 10000
E = 160000
H = 128
NC, NS = 2, 16
NW = NC * NS
EPW = E // NW
BATCH = 40
WAVE = 5
ROWS = BATCH * WAVE
NWAVES = EPW // ROWS
NCHUNK = EPW // BATCH
STRIPE = 640
NPAD = NS * STRIPE

f32 = jnp.float32
bf16 = jnp.bfloat16

EB = 2000
NB = 2000


def _dot(x, w):
    return jnp.dot(x.astype(bf16), w, preferred_element_type=f32)


def _tail(pre, w1, b1, w2, b2, w3, b3, g, beta):
    h = jnp.maximum(pre, 0.0)
    h = jnp.maximum(_dot(h, w1[...]) + b1[...], 0.0)
    h = jnp.maximum(_dot(h, w2[...]) + b2[...], 0.0)
    h = _dot(h, w3[...]) + b3[...]
    if g is not None:
        mu = jnp.mean(h, axis=-1, keepdims=True)
        var = jnp.mean((h - mu) ** 2, axis=-1, keepdims=True)
        h = (h - mu) * lax.rsqrt(var + 1e-5) * g[...] + beta[...]
    return h


def _full(shape):
    return pl.BlockSpec(shape, lambda i: (0,) * len(shape))


def _rows(block, width):
    return pl.BlockSpec((block, width), lambda i: (i, 0))


def _prep(p, lay_norm):
    ws = [w.astype(bf16) for w in p['W']]
    bs = [b.reshape(1, -1) for b in p['b']]
    if lay_norm:
        return ws, bs, p['g'].reshape(1, -1), p['beta'].reshape(1, -1)
    return ws, bs, None, None


def _enc_body(x, w0, b0, w1, b1, w2, b2, w3, b3, g, beta, out):
    pre = _dot(x[...], w0[...]) + b0[...]
    out[...] = _tail(pre, w1, b1, w2, b2, w3, b3, g, beta)


def _encoder(x, p, block):
    ws, bs, g, beta = _prep(p, True)
    rows, width = x.shape
    args = [x, ws[0], bs[0], ws[1], bs[1], ws[2], bs[2], ws[3], bs[3], g, beta]
    specs = [_rows(block, width)] + [_full(a.shape) for a in args[1:]]
    return pl.pallas_call(
        _enc_body,
        grid=(rows // block,),
        in_specs=specs,
        out_specs=_rows(block, H),
        out_shape=jax.ShapeDtypeStruct((rows, H), f32),
    )(*args)


def _dec_body(x, w0, b0, w1, b1, w2, b2, w3, b3, out):
    pre = _dot(x[...], w0[...]) + b0[...]
    out[...] = _tail(pre, w1, b1, w2, b2, w3, b3, None, None)


def _decoder(x, p, block):
    ws, bs, _, _ = _prep(p, False)
    rows = x.shape[0]
    out_w = p['W'][3].shape[1]
    args = [x, ws[0], bs[0], ws[1], bs[1], ws[2], bs[2], ws[3], bs[3]]
    specs = [_rows(block, H)] + [_full(a.shape) for a in args[1:]]
    return pl.pallas_call(
        _dec_body,
        grid=(rows // block,),
        in_specs=specs,
        out_specs=_rows(block, out_w),
        out_shape=jax.ShapeDtypeStruct((rows, out_w), f32),
    )(*args)


def _ab_body(nf, w0a, w0b, a_out, b_out):
    nfb = nf[...].astype(bf16)
    a_out[...] = jnp.dot(nfb, w0a[...], preferred_element_type=f32)
    b_out[...] = jnp.dot(nfb, w0b[...], preferred_element_type=f32)


def _ab(nf, w0a, w0b):
    args = [nf, w0a, w0b]
    specs = [_rows(NB, H), _full(w0a.shape), _full(w0b.shape)]
    return pl.pallas_call(
        _ab_body,
        grid=(N // NB,),
        in_specs=specs,
        out_specs=[_rows(NB, H), _rows(NB, H)],
        out_shape=[jax.ShapeDtypeStruct((N, H), f32),
                   jax.ShapeDtypeStruct((N, H), f32)],
    )(*args)


def _edge_body(gsum, ef, w0c, b0, w1, b1, w2, b2, w3, b3, g, beta, out):
    efv = ef[...]
    pre = gsum[...] + _dot(efv, w0c[...]) + b0[...]
    out[...] = _tail(pre, w1, b1, w2, b2, w3, b3, g, beta) + efv


def _edge_mlp(gsum, ef, p):
    ws, bs, g, beta = _prep(p, True)
    w0c = ws[0][2 * H:3 * H]
    args = [gsum, ef, w0c, bs[0], ws[1], bs[1], ws[2], bs[2], ws[3], bs[3],
            g, beta]
    specs = [_rows(EB, H)] * 2 + [_full(a.shape) for a in args[2:]]
    return pl.pallas_call(
        _edge_body,
        grid=(E // EB,),
        in_specs=specs,
        out_specs=_rows(EB, H),
        out_shape=jax.ShapeDtypeStruct((E, H), f32),
    )(*args)


def _node_body(nf, a0, a1, w0a, w0b, b0, w1, b1, w2, b2, w3, b3, g, beta, out):
    nfv = nf[...]
    agg = (a0[...] + a1[...]).astype(bf16)
    pre = (_dot(nfv, w0a[...]) +
           jnp.dot(agg, w0b[...], preferred_element_type=f32) + b0[...])
    out[...] = _tail(pre, w1, b1, w2, b2, w3, b3, g, beta) + nfv


def _node_mlp(nf, a0, a1, p):
    ws, bs, g, beta = _prep(p, True)
    w0a, w0b = ws[0][:H], ws[0][H:2 * H]
    args = [nf, a0, a1, w0a, w0b, bs[0], ws[1], bs[1], ws[2], bs[2], ws[3],
            bs[3], g, beta]
    specs = [_rows(NB, H)] * 3 + [_full(a.shape) for a in args[3:]]
    return pl.pallas_call(
        _node_body,
        grid=(N // NB,),
        in_specs=specs,
        out_specs=_rows(NB, H),
        out_shape=jax.ShapeDtypeStruct((N, H), f32),
    )(*args)


GB_FULL = 39
GB_ROWS = 128
GB_TAIL = EPW - GB_FULL * GB_ROWS


@functools.cache
def _gather_sc_build():
    mesh = plsc.VectorSubcoreMesh(core_axis_name="c", subcore_axis_name="s",
                                  num_cores=NC, num_subcores=NS)
    return functools.partial(
        pl.kernel,
        out_type=jax.ShapeDtypeStruct((E, H), f32),
        mesh=mesh,
        scratch_types=[
            pltpu.VMEM((EPW,), jnp.int32),
            pltpu.VMEM((EPW,), jnp.int32),
            pltpu.VMEM((GB_ROWS, H), f32),
            pltpu.VMEM((GB_ROWS, H), f32),
            pltpu.VMEM((GB_ROWS, H), f32),
            pltpu.VMEM((GB_ROWS, H), f32),
            pltpu.SemaphoreType.DMA,
            pltpu.SemaphoreType.DMA,
            pltpu.SemaphoreType.DMA,
        ],
    )(_gather_sc_body)


def _gather_sc(a_tab, b_tab, s, r):
    return _gather_sc_build()(a_tab, b_tab, s, r)


def _gather_sc_body(a_hbm, b_hbm, s_hbm, r_hbm, g_hbm,
                    sidx, ridx, a0, b0, a1, b1, sem_g0, sem_g1, sem_w):
    cid = lax.axis_index("c")
    sid = lax.axis_index("s")
    wid = sid * NC + cid
    base = pl.multiple_of(wid * EPW, 8)
    pltpu.sync_copy(s_hbm.at[pl.ds(base, EPW)], sidx)
    pltpu.sync_copy(r_hbm.at[pl.ds(base, EPW)], ridx)

    def fire(w, n, abuf, bbuf, sem):
        o = pl.multiple_of(w * GB_ROWS, 8)
        ha = pltpu.async_copy(a_hbm.at[sidx.at[pl.ds(o, n)]],
                              abuf.at[pl.ds(0, n)], sem)
        hb = pltpu.async_copy(b_hbm.at[ridx.at[pl.ds(o, n)]],
                              bbuf.at[pl.ds(0, n)], sem)
        return ha, hb

    def add_into(abuf, bbuf, n):
        def row(i, c):
            for j in range(H // 16):
                sl = pl.ds(j * 16, 16)
                abuf[i, sl] = abuf[i, sl] + bbuf[i, sl]
            return c
        lax.fori_loop(0, n, row, 0)

    def writeback(w, n, abuf):
        off = pl.multiple_of(base + w * GB_ROWS, 8)
        return pltpu.async_copy(abuf.at[pl.ds(0, n)],
                                g_hbm.at[pl.ds(off, n)], sem_w)

    def pair(k, carry):
        w = 2 * k
        ha0, hb0 = fire(w, GB_ROWS, a0, b0, sem_g0)
        ha1, hb1 = fire(w + 1, GB_ROWS, a1, b1, sem_g1)
        ha0.wait()
        hb0.wait()
        add_into(a0, b0, GB_ROWS)
        wb0 = writeback(w, GB_ROWS, a0)
        ha1.wait()
        hb1.wait()
        add_into(a1, b1, GB_ROWS)
        wb1 = writeback(w + 1, GB_ROWS, a1)
        wb0.wait()
        wb1.wait()
        return carry

    lax.fori_loop(0, GB_FULL // 2, pair, 0)
    ha0, hb0 = fire(GB_FULL - 1, GB_ROWS, a0, b0, sem_g0)
    ha1, hb1 = fire(GB_FULL, GB_TAIL, a1, b1, sem_g1)
    ha0.wait()
    hb0.wait()
    add_into(a0, b0, GB_ROWS)
    wb0 = writeback(GB_FULL - 1, GB_ROWS, a0)
    ha1.wait()
    hb1.wait()
    add_into(a1, b1, GB_TAIL)
    wb1 = writeback(GB_FULL, GB_TAIL, a1)
    wb0.wait()
    wb1.wait()


@functools.cache
def _scatter_sc_build():
    mesh = plsc.VectorSubcoreMesh(core_axis_name="c", subcore_axis_name="s",
                                  num_cores=NC, num_subcores=NS)
    return functools.partial(
        pl.kernel,
        out_type=jax.ShapeDtypeStruct((NC, NPAD, H), f32),
        mesh=mesh,
        scratch_types=[
            pltpu.VMEM((NCHUNK, BATCH), jnp.int32),
            pltpu.VMEM((BATCH, H), f32),
            pltpu.VMEM((BATCH, H), f32),
            pltpu.VMEM_SHARED((NPAD, H), f32),
            pltpu.SemaphoreType.DMA,
            pltpu.SemaphoreType.DMA,
            pltpu.SemaphoreType.DMA,
        ],
    )(_scatter_sc_body)


def _scatter_sc(e_new, ridx3, zeros):
    return _scatter_sc_build()(e_new, ridx3, zeros)


def _scatter_sc_body(vals_hbm, ridx3_hbm, zeros_hbm, out_hbm,
                     idxv, v0, v1, acc, sem_s0, sem_s1, sem_a):
    cid = lax.axis_index("c")
    sid = lax.axis_index("s")
    wid = sid * NC + cid
    base = pl.multiple_of(wid * EPW, 8)
    pltpu.sync_copy(ridx3_hbm.at[wid], idxv)
    stripe = pl.multiple_of(sid * STRIPE, 8)
    pltpu.sync_copy(zeros_hbm, acc.at[pl.ds(stripe, STRIPE)])
    plsc.subcore_barrier()

    def stage(w, vbuf, sem):
        voff = pl.multiple_of(base + w * BATCH, 8)
        return pltpu.async_copy(vals_hbm.at[pl.ds(voff, BATCH)], vbuf, sem)

    def scat(w, vbuf):
        return pltpu.async_copy(vbuf, acc.at[idxv.at[w]], sem_a, add=True)

    def pair(k, carry):
        w = 2 * k
        st0 = stage(w, v0, sem_s0)
        st1 = stage(w + 1, v1, sem_s1)
        st0.wait()
        h0 = scat(w, v0)
        st1.wait()
        h1 = scat(w + 1, v1)
        h0.wait()
        h1.wait()
        return carry

    lax.fori_loop(0, NCHUNK // 2, pair, 0)
    if NCHUNK % 2:
        w = NCHUNK - 1
        stage(w, v0, sem_s0).wait()
        scat(w, v0).wait()
    plsc.subcore_barrier()
    pltpu.sync_copy(acc.at[pl.ds(stripe, STRIPE)],
                    out_hbm.at[cid, pl.ds(stripe, STRIPE)])


def kernel(node_attr, edge_attr, edge_index, params):
    order = jnp.argsort(edge_index[1])
    s = edge_index[0, order].astype(jnp.int32)
    r = edge_index[1, order].astype(jnp.int32)
    edge_attr = edge_attr[order]
    ridx3 = r.reshape(NW, NCHUNK, BATCH)
    zeros = jnp.zeros((STRIPE, H), f32)

    nf = _encoder(node_attr, params['node_encoder'], NB)
    ef = _encoder(edge_attr, params['edge_encoder'], EB)

    for blk in params['blocks']:
        pE, pN = blk['edge_mlp'], blk['node_mlp']
        w0 = pE['W'][0].astype(bf16)
        a_tab, b_tab = _ab(nf, w0[:H], w0[H:2 * H])
        gsum = _gather_sc(a_tab, b_tab, s, r)
        e_new = _edge_mlp(gsum, ef, pE)
        parts = _scatter_sc(e_new, ridx3, zeros)
        nf = _node_mlp(nf, parts[0, :N], parts[1, :N], pN)
        ef = e_new
    return _decoder(nf, params['decoder'], NB)


# speedup vs baseline: 1.1452x; 1.1452x over previous
import functools

import jax
import jax.numpy as jnp
from jax import lax
from jax.experimental import pallas as pl
from jax.experimental.pallas import tpu as pltpu
from jax.experimental.pallas import tpu_sc as plsc

N = 10000
E = 160000
H = 128
NC, NS = 2, 16
NW = NC * NS
EPW = E // NW
BATCH = 40
WAVE = 5
ROWS = BATCH * WAVE
NWAVES = EPW // ROWS
NCHUNK = EPW // BATCH
STRIPE = 640
NPAD = NS * STRIPE

f32 = jnp.float32
bf16 = jnp.bfloat16

EB = 2000
NB = 2000


def _dot(x, w):
    return jnp.dot(x.astype(bf16), w, preferred_element_type=f32)


def _tail(pre, w1, b1, w2, b2, w3, b3, g, beta):
    h = jnp.maximum(pre, 0.0)
    h = jnp.maximum(_dot(h, w1[...]) + b1[...], 0.0)
    h = jnp.maximum(_dot(h, w2[...]) + b2[...], 0.0)
    h = _dot(h, w3[...]) + b3[...]
    if g is not None:
        mu = jnp.mean(h, axis=-1, keepdims=True)
        var = jnp.mean((h - mu) ** 2, axis=-1, keepdims=True)
        h = (h - mu) * lax.rsqrt(var + 1e-5) * g[...] + beta[...]
    return h


def _full(shape):
    return pl.BlockSpec(shape, lambda i: (0,) * len(shape))


def _rows(block, width):
    return pl.BlockSpec((block, width), lambda i: (i, 0))


def _prep(p, lay_norm):
    ws = [w.astype(bf16) for w in p['W']]
    bs = [b.reshape(1, -1) for b in p['b']]
    if lay_norm:
        return ws, bs, p['g'].reshape(1, -1), p['beta'].reshape(1, -1)
    return ws, bs, None, None


def _enc_body(x, w0, b0, w1, b1, w2, b2, w3, b3, g, beta, out):
    pre = _dot(x[...], w0[...]) + b0[...]
    out[...] = _tail(pre, w1, b1, w2, b2, w3, b3, g, beta)


def _encoder(x, p, block):
    ws, bs, g, beta = _prep(p, True)
    rows, width = x.shape
    args = [x, ws[0], bs[0], ws[1], bs[1], ws[2], bs[2], ws[3], bs[3], g, beta]
    specs = [_rows(block, width)] + [_full(a.shape) for a in args[1:]]
    return pl.pallas_call(
        _enc_body,
        grid=(rows // block,),
        in_specs=specs,
        out_specs=_rows(block, H),
        out_shape=jax.ShapeDtypeStruct((rows, H), f32),
    )(*args)


def _dec_body(x, w0, b0, w1, b1, w2, b2, w3, b3, out):
    pre = _dot(x[...], w0[...]) + b0[...]
    out[...] = _tail(pre, w1, b1, w2, b2, w3, b3, None, None)


def _decoder(x, p, block):
    ws, bs, _, _ = _prep(p, False)
    rows = x.shape[0]
    out_w = p['W'][3].shape[1]
    args = [x, ws[0], bs[0], ws[1], bs[1], ws[2], bs[2], ws[3], bs[3]]
    specs = [_rows(block, H)] + [_full(a.shape) for a in args[1:]]
    return pl.pallas_call(
        _dec_body,
        grid=(rows // block,),
        in_specs=specs,
        out_specs=_rows(block, out_w),
        out_shape=jax.ShapeDtypeStruct((rows, out_w), f32),
    )(*args)


def _ab_body(nf, w0a, w0b, a_out, b_out):
    nfb = nf[...].astype(bf16)
    a_out[...] = jnp.dot(nfb, w0a[...], preferred_element_type=f32)
    b_out[...] = jnp.dot(nfb, w0b[...], preferred_element_type=f32)


def _ab(nf, w0a, w0b):
    args = [nf, w0a, w0b]
    specs = [_rows(NB, H), _full(w0a.shape), _full(w0b.shape)]
    return pl.pallas_call(
        _ab_body,
        grid=(N // NB,),
        in_specs=specs,
        out_specs=[_rows(NB, H), _rows(NB, H)],
        out_shape=[jax.ShapeDtypeStruct((N, H), f32),
                   jax.ShapeDtypeStruct((N, H), f32)],
    )(*args)


def _unpack_lo(x):
    return lax.bitcast_convert_type(x << 16, f32)


def _unpack_hi(x):
    return lax.bitcast_convert_type(x & jnp.int32(-65536), f32)


def _edge_body(gp, ef, w0c, b0, w1, b1, w2, b2, w3, b3, g, beta, out):
    efv = ef[...]
    gpv = gp[...]
    gsum = jnp.concatenate([_unpack_lo(gpv), _unpack_hi(gpv)], axis=-1)
    pre = gsum + _dot(efv, w0c[...]) + b0[...]
    out[...] = _tail(pre, w1, b1, w2, b2, w3, b3, g, beta) + efv


def _edge_mlp(gsum, ef, p):
    ws, bs, g, beta = _prep(p, True)
    k = jnp.arange(H // 2)
    lo = 32 * (k // 16) + k % 16
    perm = jnp.concatenate([lo, lo + 16])
    w0c = ws[0][2 * H:3 * H][:, perm]
    b0 = bs[0][:, perm]
    w1 = ws[1][perm, :]
    args = [gsum, ef, w0c, b0, w1, bs[1], ws[2], bs[2], ws[3], bs[3],
            g, beta]
    specs = ([_rows(EB, H // 2), _rows(EB, H)] +
             [_full(a.shape) for a in args[2:]])
    return pl.pallas_call(
        _edge_body,
        grid=(E // EB,),
        in_specs=specs,
        out_specs=_rows(EB, H),
        out_shape=jax.ShapeDtypeStruct((E, H), f32),
    )(*args)


def _node_body(nf, a0, a1, w0a, w0b, b0, w1, b1, w2, b2, w3, b3, g, beta, out):
    nfv = nf[...]
    agg = (a0[...] + a1[...]).astype(bf16)
    pre = (_dot(nfv, w0a[...]) +
           jnp.dot(agg, w0b[...], preferred_element_type=f32) + b0[...])
    out[...] = _tail(pre, w1, b1, w2, b2, w3, b3, g, beta) + nfv


def _node_mlp(nf, a0, a1, p):
    ws, bs, g, beta = _prep(p, True)
    w0a, w0b = ws[0][:H], ws[0][H:2 * H]
    args = [nf, a0, a1, w0a, w0b, bs[0], ws[1], bs[1], ws[2], bs[2], ws[3],
            bs[3], g, beta]
    specs = [_rows(NB, H)] * 3 + [_full(a.shape) for a in args[3:]]
    return pl.pallas_call(
        _node_body,
        grid=(N // NB,),
        in_specs=specs,
        out_specs=_rows(NB, H),
        out_shape=jax.ShapeDtypeStruct((N, H), f32),
    )(*args)


GB_FULL = 39
GB_ROWS = 128
GB_TAIL = EPW - GB_FULL * GB_ROWS


@functools.cache
def _gather_sc_build():
    mesh = plsc.VectorSubcoreMesh(core_axis_name="c", subcore_axis_name="s",
                                  num_cores=NC, num_subcores=NS)
    return functools.partial(
        pl.kernel,
        out_type=jax.ShapeDtypeStruct((E, H // 2), jnp.int32),
        mesh=mesh,
        scratch_types=[
            pltpu.VMEM((EPW,), jnp.int32),
            pltpu.VMEM((EPW,), jnp.int32),
            pltpu.VMEM((GB_ROWS, H), f32),
            pltpu.VMEM((GB_ROWS, H), f32),
            pltpu.VMEM((GB_ROWS, H), f32),
            pltpu.VMEM((GB_ROWS, H), f32),
            pltpu.VMEM((GB_ROWS, H // 2), jnp.int32),
            pltpu.VMEM((GB_ROWS, H // 2), jnp.int32),
            pltpu.SemaphoreType.DMA,
            pltpu.SemaphoreType.DMA,
            pltpu.SemaphoreType.DMA,
        ],
    )(_gather_sc_body)


def _gather_sc(a_tab, b_tab, s, r):
    return _gather_sc_build()(a_tab, b_tab, s, r)


def _gather_sc_body(a_hbm, b_hbm, s_hbm, r_hbm, g_hbm,
                    sidx, ridx, a0, b0, a1, b1, g0, g1,
                    sem_g0, sem_g1, sem_w):
    cid = lax.axis_index("c")
    sid = lax.axis_index("s")
    wid = sid * NC + cid
    base = pl.multiple_of(wid * EPW, 8)
    pltpu.sync_copy(s_hbm.at[pl.ds(base, EPW)], sidx)
    pltpu.sync_copy(r_hbm.at[pl.ds(base, EPW)], ridx)

    def fire(w, n, abuf, bbuf, sem):
        o = pl.multiple_of(w * GB_ROWS, 8)
        ha = pltpu.async_copy(a_hbm.at[sidx.at[pl.ds(o, n)]],
                              abuf.at[pl.ds(0, n)], sem)
        hb = pltpu.async_copy(b_hbm.at[ridx.at[pl.ds(o, n)]],
                              bbuf.at[pl.ds(0, n)], sem)
        return ha, hb

    def add_pack(abuf, bbuf, gbuf, n):
        def row(i, c):
            for j in range(H // 32):
                lo = abuf[i, pl.ds(32 * j, 16)] + bbuf[i, pl.ds(32 * j, 16)]
                hi = (abuf[i, pl.ds(32 * j + 16, 16)] +
                      bbuf[i, pl.ds(32 * j + 16, 16)])
                lob = lax.bitcast_convert_type(lo, jnp.int32)
                hib = lax.bitcast_convert_type(hi, jnp.int32)
                gbuf[i, pl.ds(16 * j, 16)] = (
                    lax.shift_right_logical(lob, 16) |
                    (hib & jnp.int32(-65536)))
            return c
        lax.fori_loop(0, n, row, 0)

    def writeback(w, n, gbuf):
        off = pl.multiple_of(base + w * GB_ROWS, 8)
        return pltpu.async_copy(gbuf.at[pl.ds(0, n)],
                                g_hbm.at[pl.ds(off, n)], sem_w)

    def pair(k, carry):
        w = 2 * k
        ha0, hb0 = fire(w, GB_ROWS, a0, b0, sem_g0)
        ha1, hb1 = fire(w + 1, GB_ROWS, a1, b1, sem_g1)
        ha0.wait()
        hb0.wait()
        add_pack(a0, b0, g0, GB_ROWS)
        wb0 = writeback(w, GB_ROWS, g0)
        ha1.wait()
        hb1.wait()
        add_pack(a1, b1, g1, GB_ROWS)
        wb1 = writeback(w + 1, GB_ROWS, g1)
        wb0.wait()
        wb1.wait()
        return carry

    lax.fori_loop(0, GB_FULL // 2, pair, 0)
    ha0, hb0 = fire(GB_FULL - 1, GB_ROWS, a0, b0, sem_g0)
    ha1, hb1 = fire(GB_FULL, GB_TAIL, a1, b1, sem_g1)
    ha0.wait()
    hb0.wait()
    add_pack(a0, b0, g0, GB_ROWS)
    wb0 = writeback(GB_FULL - 1, GB_ROWS, g0)
    ha1.wait()
    hb1.wait()
    add_pack(a1, b1, g1, GB_TAIL)
    wb1 = writeback(GB_FULL, GB_TAIL, g1)
    wb0.wait()
    wb1.wait()


@functools.cache
def _scatter_sc_build():
    mesh = plsc.VectorSubcoreMesh(core_axis_name="c", subcore_axis_name="s",
                                  num_cores=NC, num_subcores=NS)
    return functools.partial(
        pl.kernel,
        out_type=jax.ShapeDtypeStruct((NC, NPAD, H), f32),
        mesh=mesh,
        scratch_types=[
            pltpu.VMEM((NCHUNK, BATCH), jnp.int32),
            pltpu.VMEM((BATCH, H), f32),
            pltpu.VMEM((BATCH, H), f32),
            pltpu.VMEM_SHARED((NPAD, H), f32),
            pltpu.SemaphoreType.DMA,
            pltpu.SemaphoreType.DMA,
            pltpu.SemaphoreType.DMA,
        ],
    )(_scatter_sc_body)


def _scatter_sc(e_new, ridx3, zeros):
    return _scatter_sc_build()(e_new, ridx3, zeros)


def _scatter_sc_body(vals_hbm, ridx3_hbm, zeros_hbm, out_hbm,
                     idxv, v0, v1, acc, sem_s0, sem_s1, sem_a):
    cid = lax.axis_index("c")
    sid = lax.axis_index("s")
    wid = sid * NC + cid
    base = pl.multiple_of(wid * EPW, 8)
    pltpu.sync_copy(ridx3_hbm.at[wid], idxv)
    stripe = pl.multiple_of(sid * STRIPE, 8)
    pltpu.sync_copy(zeros_hbm, acc.at[pl.ds(stripe, STRIPE)])
    plsc.subcore_barrier()

    def stage(w, vbuf, sem):
        voff = pl.multiple_of(base + w * BATCH, 8)
        return pltpu.async_copy(vals_hbm.at[pl.ds(voff, BATCH)], vbuf, sem)

    def scat(w, vbuf):
        return pltpu.async_copy(vbuf, acc.at[idxv.at[w]], sem_a, add=True)

    def pair(k, carry):
        w = 2 * k
        st0 = stage(w, v0, sem_s0)
        st1 = stage(w + 1, v1, sem_s1)
        st0.wait()
        h0 = scat(w, v0)
        st1.wait()
        h1 = scat(w + 1, v1)
        h0.wait()
        h1.wait()
        return carry

    lax.fori_loop(0, NCHUNK // 2, pair, 0)
    if NCHUNK % 2:
        w = NCHUNK - 1
        stage(w, v0, sem_s0).wait()
        scat(w, v0).wait()
    plsc.subcore_barrier()
    pltpu.sync_copy(acc.at[pl.ds(stripe, STRIPE)],
                    out_hbm.at[cid, pl.ds(stripe, STRIPE)])


def kernel(node_attr, edge_attr, edge_index, params):
    s = edge_index[0].astype(jnp.int32)
    r = edge_index[1].astype(jnp.int32)
    ridx3 = r.reshape(NW, NCHUNK, BATCH)
    zeros = jnp.zeros((STRIPE, H), f32)

    nf = _encoder(node_attr, params['node_encoder'], NB)
    ef = _encoder(edge_attr, params['edge_encoder'], EB)

    for blk in params['blocks']:
        pE, pN = blk['edge_mlp'], blk['node_mlp']
        w0 = pE['W'][0].astype(bf16)
        a_tab, b_tab = _ab(nf, w0[:H], w0[H:2 * H])
        gsum = _gather_sc(a_tab, b_tab, s, r)
        e_new = _edge_mlp(gsum, ef, pE)
        parts = _scatter_sc(e_new, ridx3, zeros)
        nf = _node_mlp(nf, parts[0, :N], parts[1, :N], pN)
        ef = e_new
    return _decoder(nf, params['decoder'], NB)


# speedup vs baseline: 1.2978x; 1.1333x over previous
import functools

import jax
import jax.numpy as jnp
from jax import lax
from jax.experimental import pallas as pl
from jax.experimental.pallas import tpu as pltpu
from jax.experimental.pallas import tpu_sc as plsc

N = 10000
E = 160000
H = 128
NC, NS = 2, 16
NW = NC * NS
EPW = E // NW
BATCH = 40
WAVE = 5
ROWS = BATCH * WAVE
NWAVES = EPW // ROWS
NCHUNK = EPW // BATCH
STRIPE = 640
NPAD = NS * STRIPE

f32 = jnp.float32
bf16 = jnp.bfloat16

EB = 2000
NB = 2000


def _dot(x, w):
    return jnp.dot(x.astype(bf16), w, preferred_element_type=f32)


def _tail(pre, w1, b1, w2, b2, w3, b3, g, beta):
    h = jnp.maximum(pre, 0.0)
    h = jnp.maximum(_dot(h, w1[...]) + b1[...], 0.0)
    h = jnp.maximum(_dot(h, w2[...]) + b2[...], 0.0)
    h = _dot(h, w3[...]) + b3[...]
    if g is not None:
        mu = jnp.mean(h, axis=-1, keepdims=True)
        var = jnp.mean((h - mu) ** 2, axis=-1, keepdims=True)
        h = (h - mu) * lax.rsqrt(var + 1e-5) * g[...] + beta[...]
    return h


def _full(shape):
    return pl.BlockSpec(shape, lambda i: (0,) * len(shape))


def _rows(block, width):
    return pl.BlockSpec((block, width), lambda i: (i, 0))


def _prep(p, lay_norm):
    ws = [w.astype(bf16) for w in p['W']]
    bs = [b.reshape(1, -1) for b in p['b']]
    if lay_norm:
        return ws, bs, p['g'].reshape(1, -1), p['beta'].reshape(1, -1)
    return ws, bs, None, None


def _enc_body(x, w0, b0, w1, b1, w2, b2, w3, b3, g, beta, out):
    pre = _dot(x[...], w0[...]) + b0[...]
    out[...] = _tail(pre, w1, b1, w2, b2, w3, b3, g, beta)


def _encoder(x, p, block):
    ws, bs, g, beta = _prep(p, True)
    rows, width = x.shape
    args = [x, ws[0], bs[0], ws[1], bs[1], ws[2], bs[2], ws[3], bs[3], g, beta]
    specs = [_rows(block, width)] + [_full(a.shape) for a in args[1:]]
    return pl.pallas_call(
        _enc_body,
        grid=(rows // block,),
        in_specs=specs,
        out_specs=_rows(block, H),
        out_shape=jax.ShapeDtypeStruct((rows, H), f32),
    )(*args)


def _dec_body(x, w0, b0, w1, b1, w2, b2, w3, b3, out):
    pre = _dot(x[...], w0[...]) + b0[...]
    out[...] = _tail(pre, w1, b1, w2, b2, w3, b3, None, None)


def _decoder(x, p, block):
    ws, bs, _, _ = _prep(p, False)
    rows = x.shape[0]
    out_w = p['W'][3].shape[1]
    args = [x, ws[0], bs[0], ws[1], bs[1], ws[2], bs[2], ws[3], bs[3]]
    specs = [_rows(block, H)] + [_full(a.shape) for a in args[1:]]
    return pl.pallas_call(
        _dec_body,
        grid=(rows // block,),
        in_specs=specs,
        out_specs=_rows(block, out_w),
        out_shape=jax.ShapeDtypeStruct((rows, out_w), f32),
    )(*args)


def _ab_body(nf, w0a, w0b, a_out, b_out):
    nfb = nf[...].astype(bf16)
    a_out[...] = jnp.dot(nfb, w0a[...], preferred_element_type=f32)
    b_out[...] = jnp.dot(nfb, w0b[...], preferred_element_type=f32)


def _ab(nf, w0a, w0b):
    args = [nf, w0a, w0b]
    specs = [_rows(NB, H), _full(w0a.shape), _full(w0b.shape)]
    return pl.pallas_call(
        _ab_body,
        grid=(N // NB,),
        in_specs=specs,
        out_specs=[_rows(NB, H), _rows(NB, H)],
        out_shape=[jax.ShapeDtypeStruct((N, H), f32),
                   jax.ShapeDtypeStruct((N, H), f32)],
    )(*args)


def _unpack_lo(x):
    return lax.bitcast_convert_type(x << 16, f32)


def _unpack_hi(x):
    return lax.bitcast_convert_type(x & jnp.int32(-65536), f32)


def _edge_body(gp, ef, w0c, b0, w1, b1, w2, b2, w3, b3, g, beta, out):
    efv = ef[...]
    gpv = gp[...]
    gsum = jnp.concatenate([_unpack_lo(gpv), _unpack_hi(gpv)], axis=-1)
    pre = gsum + _dot(efv, w0c[...]) + b0[...]
    out[...] = _tail(pre, w1, b1, w2, b2, w3, b3, g, beta) + efv


def _edge_mlp(gsum, ef, p):
    ws, bs, g, beta = _prep(p, True)
    k = jnp.arange(H // 2)
    lo = 32 * (k // 16) + k % 16
    perm = jnp.concatenate([lo, lo + 16])
    w0c = ws[0][2 * H:3 * H][:, perm]
    b0 = bs[0][:, perm]
    w1 = ws[1][perm, :]
    args = [gsum, ef, w0c, b0, w1, bs[1], ws[2], bs[2], ws[3], bs[3],
            g, beta]
    specs = ([_rows(EB, H // 2), _rows(EB, H)] +
             [_full(a.shape) for a in args[2:]])
    return pl.pallas_call(
        _edge_body,
        grid=(E // EB,),
        in_specs=specs,
        out_specs=_rows(EB, H),
        out_shape=jax.ShapeDtypeStruct((E, H), f32),
    )(*args)


def _node_body(nf, a0, a1, w0a, w0b, b0, w1, b1, w2, b2, w3, b3, g, beta, out):
    nfv = nf[...]
    agg = (a0[...] + a1[...]).astype(bf16)
    pre = (_dot(nfv, w0a[...]) +
           jnp.dot(agg, w0b[...], preferred_element_type=f32) + b0[...])
    out[...] = _tail(pre, w1, b1, w2, b2, w3, b3, g, beta) + nfv


def _node_mlp(nf, a0, a1, p):
    ws, bs, g, beta = _prep(p, True)
    w0a, w0b = ws[0][:H], ws[0][H:2 * H]
    args = [nf, a0, a1, w0a, w0b, bs[0], ws[1], bs[1], ws[2], bs[2], ws[3],
            bs[3], g, beta]
    specs = [_rows(NB, H)] * 3 + [_full(a.shape) for a in args[3:]]
    return pl.pallas_call(
        _node_body,
        grid=(N // NB,),
        in_specs=specs,
        out_specs=_rows(NB, H),
        out_shape=jax.ShapeDtypeStruct((N, H), f32),
    )(*args)


GB_FULL = 39
GB_ROWS = 128
GB_TAIL = EPW - GB_FULL * GB_ROWS


@functools.cache
def _gather_sc_build():
    mesh = plsc.VectorSubcoreMesh(core_axis_name="c", subcore_axis_name="s",
                                  num_cores=NC, num_subcores=NS)
    return functools.partial(
        pl.kernel,
        out_type=jax.ShapeDtypeStruct((E, H // 2), jnp.int32),
        mesh=mesh,
        scratch_types=[
            pltpu.VMEM((EPW,), jnp.int32),
            pltpu.VMEM((EPW,), jnp.int32),
            pltpu.VMEM((GB_ROWS, H), f32),
            pltpu.VMEM((GB_ROWS, H), f32),
            pltpu.VMEM((GB_ROWS, H), f32),
            pltpu.VMEM((GB_ROWS, H), f32),
            pltpu.VMEM((GB_ROWS, H // 2), jnp.int32),
            pltpu.VMEM((GB_ROWS, H // 2), jnp.int32),
            pltpu.SemaphoreType.DMA,
            pltpu.SemaphoreType.DMA,
            pltpu.SemaphoreType.DMA,
        ],
    )(_gather_sc_body)


def _gather_sc(a_tab, b_tab, s, r):
    return _gather_sc_build()(a_tab, b_tab, s, r)


def _gather_sc_body(a_hbm, b_hbm, s_hbm, r_hbm, g_hbm,
                    sidx, ridx, a0, b0, a1, b1, g0, g1,
                    sem_g0, sem_g1, sem_w):
    cid = lax.axis_index("c")
    sid = lax.axis_index("s")
    wid = sid * NC + cid
    base = pl.multiple_of(wid * EPW, 8)
    pltpu.sync_copy(s_hbm.at[pl.ds(base, EPW)], sidx)
    pltpu.sync_copy(r_hbm.at[pl.ds(base, EPW)], ridx)

    def fire(w, n, abuf, bbuf, sem):
        o = pl.multiple_of(w * GB_ROWS, 8)
        ha = pltpu.async_copy(a_hbm.at[sidx.at[pl.ds(o, n)]],
                              abuf.at[pl.ds(0, n)], sem)
        hb = pltpu.async_copy(b_hbm.at[ridx.at[pl.ds(o, n)]],
                              bbuf.at[pl.ds(0, n)], sem)
        return ha, hb

    def add_pack(abuf, bbuf, gbuf, n):
        def row(i, c):
            for j in range(H // 32):
                lo = abuf[i, pl.ds(32 * j, 16)] + bbuf[i, pl.ds(32 * j, 16)]
                hi = (abuf[i, pl.ds(32 * j + 16, 16)] +
                      bbuf[i, pl.ds(32 * j + 16, 16)])
                lob = lax.bitcast_convert_type(lo, jnp.int32)
                hib = lax.bitcast_convert_type(hi, jnp.int32)
                gbuf[i, pl.ds(16 * j, 16)] = (
                    lax.shift_right_logical(lob, 16) |
                    (hib & jnp.int32(-65536)))
            return c
        lax.fori_loop(0, n, row, 0)

    def writeback(w, n, gbuf):
        off = pl.multiple_of(base + w * GB_ROWS, 8)
        return pltpu.async_copy(gbuf.at[pl.ds(0, n)],
                                g_hbm.at[pl.ds(off, n)], sem_w)

    def pair(k, carry):
        w = 2 * k
        ha0, hb0 = fire(w, GB_ROWS, a0, b0, sem_g0)
        ha1, hb1 = fire(w + 1, GB_ROWS, a1, b1, sem_g1)
        ha0.wait()
        hb0.wait()
        add_pack(a0, b0, g0, GB_ROWS)
        wb0 = writeback(w, GB_ROWS, g0)
        ha1.wait()
        hb1.wait()
        add_pack(a1, b1, g1, GB_ROWS)
        wb1 = writeback(w + 1, GB_ROWS, g1)
        wb0.wait()
        wb1.wait()
        return carry

    lax.fori_loop(0, GB_FULL // 2, pair, 0)
    ha0, hb0 = fire(GB_FULL - 1, GB_ROWS, a0, b0, sem_g0)
    ha1, hb1 = fire(GB_FULL, GB_TAIL, a1, b1, sem_g1)
    ha0.wait()
    hb0.wait()
    add_pack(a0, b0, g0, GB_ROWS)
    wb0 = writeback(GB_FULL - 1, GB_ROWS, g0)
    ha1.wait()
    hb1.wait()
    add_pack(a1, b1, g1, GB_TAIL)
    wb1 = writeback(GB_FULL, GB_TAIL, g1)
    wb0.wait()
    wb1.wait()


@functools.cache
def _scatter_sc_build():
    mesh = plsc.VectorSubcoreMesh(core_axis_name="c", subcore_axis_name="s",
                                  num_cores=NC, num_subcores=NS)
    return functools.partial(
        pl.kernel,
        out_type=jax.ShapeDtypeStruct((NC, NPAD, H), f32),
        mesh=mesh,
        scratch_types=[
            pltpu.VMEM((GB_FULL, GB_ROWS), jnp.int32),
            pltpu.VMEM((GB_TAIL,), jnp.int32),
            pltpu.VMEM((GB_ROWS, H), f32),
            pltpu.VMEM((GB_ROWS, H), f32),
            pltpu.VMEM_SHARED((NPAD, H), f32),
            pltpu.SemaphoreType.DMA,
            pltpu.SemaphoreType.DMA,
            pltpu.SemaphoreType.DMA,
        ],
    )(_scatter_sc_body)


def _scatter_sc(e_new, rmain, rtail, zeros):
    return _scatter_sc_build()(e_new, rmain, rtail, zeros)


def _scatter_sc_body(vals_hbm, rmain_hbm, rtail_hbm, zeros_hbm, out_hbm,
                     idxm, idxt, v0, v1, acc, sem_s0, sem_s1, sem_a):
    cid = lax.axis_index("c")
    sid = lax.axis_index("s")
    wid = sid * NC + cid
    base = pl.multiple_of(wid * EPW, 8)
    pltpu.sync_copy(rmain_hbm.at[wid], idxm)
    pltpu.sync_copy(rtail_hbm.at[wid], idxt)
    stripe = sid * STRIPE
    pltpu.sync_copy(zeros_hbm, acc.at[pl.ds(stripe, STRIPE)])
    plsc.subcore_barrier()

    def stage(w, n, vbuf, sem):
        voff = pl.multiple_of(base + w * GB_ROWS, 8)
        return pltpu.async_copy(vals_hbm.at[pl.ds(voff, n)],
                                vbuf.at[pl.ds(0, n)], sem)

    def pair(k, carry):
        w = 2 * k
        st0 = stage(w, GB_ROWS, v0, sem_s0)
        st1 = stage(w + 1, GB_ROWS, v1, sem_s1)
        st0.wait()
        h0 = pltpu.async_copy(v0, acc.at[idxm.at[w]], sem_a, add=True)
        st1.wait()
        h1 = pltpu.async_copy(v1, acc.at[idxm.at[w + 1]], sem_a, add=True)
        h0.wait()
        h1.wait()
        return carry

    lax.fori_loop(0, GB_FULL // 2, pair, 0)
    st0 = stage(GB_FULL - 1, GB_ROWS, v0, sem_s0)
    st1 = stage(GB_FULL, GB_TAIL, v1, sem_s1)
    st0.wait()
    h0 = pltpu.async_copy(v0, acc.at[idxm.at[GB_FULL - 1]], sem_a, add=True)
    st1.wait()
    h1 = pltpu.async_copy(v1.at[pl.ds(0, GB_TAIL)], acc.at[idxt], sem_a,
                          add=True)
    h0.wait()
    h1.wait()
    plsc.subcore_barrier()
    pltpu.sync_copy(acc.at[pl.ds(stripe, STRIPE)],
                    out_hbm.at[cid, pl.ds(stripe, STRIPE)])


def kernel(node_attr, edge_attr, edge_index, params):
    s = edge_index[0].astype(jnp.int32)
    r = edge_index[1].astype(jnp.int32)
    rr = r.reshape(NW, EPW)
    rmain = rr[:, :GB_FULL * GB_ROWS].reshape(NW, GB_FULL, GB_ROWS)
    rtail = rr[:, GB_FULL * GB_ROWS:]
    zeros = jnp.zeros((STRIPE, H), f32)

    nf = _encoder(node_attr, params['node_encoder'], NB)
    ef = _encoder(edge_attr, params['edge_encoder'], EB)

    for blk in params['blocks']:
        pE, pN = blk['edge_mlp'], blk['node_mlp']
        w0 = pE['W'][0].astype(bf16)
        a_tab, b_tab = _ab(nf, w0[:H], w0[H:2 * H])
        gsum = _gather_sc(a_tab, b_tab, s, r)
        e_new = _edge_mlp(gsum, ef, pE)
        parts = _scatter_sc(e_new, rmain, rtail, zeros)
        nf = _node_mlp(nf, parts[0, :N], parts[1, :N], pN)
        ef = e_new
    return _decoder(nf, params['decoder'], NB)


# speedup vs baseline: 1.3207x; 1.0176x over previous
import functools

import jax
import jax.numpy as jnp
from jax import lax
from jax.experimental import pallas as pl
from jax.experimental.pallas import tpu as pltpu
from jax.experimental.pallas import tpu_sc as plsc

N = 10000
E = 160000
H = 128
NC, NS = 2, 16
NW = NC * NS
EPW = E // NW
BATCH = 40
WAVE = 5
ROWS = BATCH * WAVE
NWAVES = EPW // ROWS
NCHUNK = EPW // BATCH
STRIPE = 640
NPAD = NS * STRIPE

f32 = jnp.float32
bf16 = jnp.bfloat16

EB = 2000
NB = 2000


def _dot(x, w):
    return jnp.dot(x.astype(bf16), w, preferred_element_type=f32)


def _tail(pre, w1, b1, w2, b2, w3, b3, g, beta):
    h = jnp.maximum(pre, 0.0)
    h = jnp.maximum(_dot(h, w1[...]) + b1[...], 0.0)
    h = jnp.maximum(_dot(h, w2[...]) + b2[...], 0.0)
    h = _dot(h, w3[...]) + b3[...]
    if g is not None:
        mu = jnp.mean(h, axis=-1, keepdims=True)
        var = jnp.mean((h - mu) ** 2, axis=-1, keepdims=True)
        h = (h - mu) * lax.rsqrt(var + 1e-5) * g[...] + beta[...]
    return h


def _full(shape):
    return pl.BlockSpec(shape, lambda i: (0,) * len(shape))


def _rows(block, width):
    return pl.BlockSpec((block, width), lambda i: (i, 0))


def _prep(p, lay_norm):
    ws = [w.astype(bf16) for w in p['W']]
    bs = [b.reshape(1, -1) for b in p['b']]
    if lay_norm:
        return ws, bs, p['g'].reshape(1, -1), p['beta'].reshape(1, -1)
    return ws, bs, None, None


def _enc_body(x, w0, b0, w1, b1, w2, b2, w3, b3, g, beta, out):
    pre = _dot(x[...], w0[...]) + b0[...]
    out[...] = _tail(pre, w1, b1, w2, b2, w3, b3, g, beta)


def _encoder(x, p, block):
    ws, bs, g, beta = _prep(p, True)
    rows, width = x.shape
    args = [x, ws[0], bs[0], ws[1], bs[1], ws[2], bs[2], ws[3], bs[3], g, beta]
    specs = [_rows(block, width)] + [_full(a.shape) for a in args[1:]]
    return pl.pallas_call(
        _enc_body,
        grid=(rows // block,),
        in_specs=specs,
        out_specs=_rows(block, H),
        out_shape=jax.ShapeDtypeStruct((rows, H), f32),
    )(*args)


def _dec_body(x, w0, b0, w1, b1, w2, b2, w3, b3, out):
    pre = _dot(x[...], w0[...]) + b0[...]
    out[...] = _tail(pre, w1, b1, w2, b2, w3, b3, None, None)


def _decoder(x, p, block):
    ws, bs, _, _ = _prep(p, False)
    rows = x.shape[0]
    out_w = p['W'][3].shape[1]
    args = [x, ws[0], bs[0], ws[1], bs[1], ws[2], bs[2], ws[3], bs[3]]
    specs = [_rows(block, H)] + [_full(a.shape) for a in args[1:]]
    return pl.pallas_call(
        _dec_body,
        grid=(rows // block,),
        in_specs=specs,
        out_specs=_rows(block, out_w),
        out_shape=jax.ShapeDtypeStruct((rows, out_w), f32),
    )(*args)


def _ab_body(nf, w0a, w0b, a_out, b_out):
    nfb = nf[...].astype(bf16)
    a_out[...] = jnp.dot(nfb, w0a[...], preferred_element_type=f32)
    b_out[...] = jnp.dot(nfb, w0b[...], preferred_element_type=f32)


def _ab(nf, w0a, w0b):
    args = [nf, w0a, w0b]
    specs = [_rows(NB, H), _full(w0a.shape), _full(w0b.shape)]
    return pl.pallas_call(
        _ab_body,
        grid=(N // NB,),
        in_specs=specs,
        out_specs=[_rows(NB, H), _rows(NB, H)],
        out_shape=[jax.ShapeDtypeStruct((N, H), f32),
                   jax.ShapeDtypeStruct((N, H), f32)],
    )(*args)


def _unpack_lo(x):
    return lax.bitcast_convert_type(x << 16, f32)


def _unpack_hi(x):
    return lax.bitcast_convert_type(x & jnp.int32(-65536), f32)


def _edge_body(gp, ef, w0c, b0, w1, b1, w2, b2, w3, b3, g, beta, out):
    efv = ef[...]
    gpv = gp[...]
    gsum = jnp.concatenate([_unpack_lo(gpv), _unpack_hi(gpv)], axis=-1)
    pre = gsum + _dot(efv, w0c[...]) + b0[...]
    out[...] = _tail(pre, w1, b1, w2, b2, w3, b3, g, beta) + efv


def _edge_mlp(gsum, ef, p):
    ws, bs, g, beta = _prep(p, True)
    k = jnp.arange(H // 2)
    lo = 32 * (k // 16) + k % 16
    perm = jnp.concatenate([lo, lo + 16])
    w0c = ws[0][2 * H:3 * H][:, perm]
    b0 = bs[0][:, perm]
    w1 = ws[1][perm, :]
    args = [gsum, ef, w0c, b0, w1, bs[1], ws[2], bs[2], ws[3], bs[3],
            g, beta]
    specs = ([_rows(EB, H // 2), _rows(EB, H)] +
             [_full(a.shape) for a in args[2:]])
    return pl.pallas_call(
        _edge_body,
        grid=(E // EB,),
        in_specs=specs,
        out_specs=_rows(EB, H),
        out_shape=jax.ShapeDtypeStruct((E, H), f32),
    )(*args)


def _node_body(nf, a0, a1, w0a, w0b, b0, w1, b1, w2, b2, w3, b3, g, beta, out):
    nfv = nf[...]
    agg = (a0[...] + a1[...]).astype(bf16)
    pre = (_dot(nfv, w0a[...]) +
           jnp.dot(agg, w0b[...], preferred_element_type=f32) + b0[...])
    out[...] = _tail(pre, w1, b1, w2, b2, w3, b3, g, beta) + nfv


def _node_mlp(nf, a0, a1, p):
    ws, bs, g, beta = _prep(p, True)
    w0a, w0b = ws[0][:H], ws[0][H:2 * H]
    args = [nf, a0, a1, w0a, w0b, bs[0], ws[1], bs[1], ws[2], bs[2], ws[3],
            bs[3], g, beta]
    specs = [_rows(NB, H)] * 3 + [_full(a.shape) for a in args[3:]]
    return pl.pallas_call(
        _node_body,
        grid=(N // NB,),
        in_specs=specs,
        out_specs=_rows(NB, H),
        out_shape=jax.ShapeDtypeStruct((N, H), f32),
    )(*args)


GB_FULL = 39
GB_ROWS = 128
GB_TAIL = EPW - GB_FULL * GB_ROWS


@functools.cache
def _gather_sc_build():
    mesh = plsc.VectorSubcoreMesh(core_axis_name="c", subcore_axis_name="s",
                                  num_cores=NC, num_subcores=NS)
    return functools.partial(
        pl.kernel,
        out_type=jax.ShapeDtypeStruct((E, H // 2), jnp.int32),
        mesh=mesh,
        scratch_types=[
            pltpu.VMEM((EPW,), jnp.int32),
            pltpu.VMEM((EPW,), jnp.int32),
            pltpu.VMEM((GB_ROWS, H), f32),
            pltpu.VMEM((GB_ROWS, H), f32),
            pltpu.VMEM((GB_ROWS, H), f32),
            pltpu.VMEM((GB_ROWS, H), f32),
            pltpu.VMEM((GB_ROWS, H // 2), jnp.int32),
            pltpu.VMEM((GB_ROWS, H // 2), jnp.int32),
            pltpu.SemaphoreType.DMA,
            pltpu.SemaphoreType.DMA,
            pltpu.SemaphoreType.DMA,
            pltpu.SemaphoreType.DMA,
        ],
    )(_gather_sc_body)


def _gather_sc(a_tab, b_tab, s, r):
    return _gather_sc_build()(a_tab, b_tab, s, r)


def _gather_sc_body(a_hbm, b_hbm, s_hbm, r_hbm, g_hbm,
                    sidx, ridx, a0, b0, a1, b1, g0, g1,
                    sem_g0, sem_g1, sem_w0, sem_w1):
    cid = lax.axis_index("c")
    sid = lax.axis_index("s")
    wid = sid * NC + cid
    base = pl.multiple_of(wid * EPW, 8)
    pltpu.sync_copy(s_hbm.at[pl.ds(base, EPW)], sidx)
    pltpu.sync_copy(r_hbm.at[pl.ds(base, EPW)], ridx)

    abufs, bbufs, gbufs = (a0, a1), (b0, b1), (g0, g1)
    gsems, wsems = (sem_g0, sem_g1), (sem_w0, sem_w1)

    def fire(w, n, slot):
        o = pl.multiple_of(w * GB_ROWS, 8)
        pltpu.async_copy(a_hbm.at[sidx.at[pl.ds(o, n)]],
                         abufs[slot].at[pl.ds(0, n)], gsems[slot])
        pltpu.async_copy(b_hbm.at[ridx.at[pl.ds(o, n)]],
                         bbufs[slot].at[pl.ds(0, n)], gsems[slot])

    def wait_g(n, slot):
        pltpu.make_async_copy(a_hbm.at[pl.ds(0, n)],
                              abufs[slot].at[pl.ds(0, n)], gsems[slot]).wait()
        pltpu.make_async_copy(b_hbm.at[pl.ds(0, n)],
                              bbufs[slot].at[pl.ds(0, n)], gsems[slot]).wait()

    def wait_wb(slot):
        pltpu.make_async_copy(g_hbm.at[pl.ds(0, GB_ROWS)],
                              gbufs[slot], wsems[slot]).wait()

    def pack(n, slot):
        abuf, bbuf, gbuf = abufs[slot], bbufs[slot], gbufs[slot]

        def rnd(bits):
            return bits + 32767 + (lax.shift_right_logical(bits, 16) & 1)

        def row(i, c):
            for j in range(H // 32):
                lo = abuf[i, pl.ds(32 * j, 16)] + bbuf[i, pl.ds(32 * j, 16)]
                hi = (abuf[i, pl.ds(32 * j + 16, 16)] +
                      bbuf[i, pl.ds(32 * j + 16, 16)])
                lob = rnd(lax.bitcast_convert_type(lo, jnp.int32))
                hib = rnd(lax.bitcast_convert_type(hi, jnp.int32))
                gbuf[i, pl.ds(16 * j, 16)] = (
                    lax.shift_right_logical(lob, 16) |
                    (hib & jnp.int32(-65536)))
            return c
        lax.fori_loop(0, n, row, 0)

    def fire_wb(w, n, slot):
        off = pl.multiple_of(base + w * GB_ROWS, 8)
        return pltpu.async_copy(gbufs[slot].at[pl.ds(0, n)],
                                g_hbm.at[pl.ds(off, n)], wsems[slot])

    fire(0, GB_ROWS, 0)
    fire(1, GB_ROWS, 1)
    for w in (0, 1):
        wait_g(GB_ROWS, w)
        pack(GB_ROWS, w)
        fire_wb(w, GB_ROWS, w)
        fire(w + 2, GB_ROWS, w)

    def step(w, slot):
        wait_g(GB_ROWS, slot)
        wait_wb(slot)
        pack(GB_ROWS, slot)
        fire_wb(w, GB_ROWS, slot)
        fire(w + 2, GB_ROWS, slot)

    def body(k, carry):
        step(2 * k, 0)
        step(2 * k + 1, 1)
        return carry

    lax.fori_loop(1, 18, body, 0)
    step(36, 0)
    wait_g(GB_ROWS, 1)
    wait_wb(1)
    pack(GB_ROWS, 1)
    fire_wb(37, GB_ROWS, 1)
    fire(GB_FULL, GB_TAIL, 1)
    wait_g(GB_ROWS, 0)
    wait_wb(0)
    pack(GB_ROWS, 0)
    wb38 = fire_wb(38, GB_ROWS, 0)
    wait_g(GB_TAIL, 1)
    wait_wb(1)
    pack(GB_TAIL, 1)
    wb39 = fire_wb(GB_FULL, GB_TAIL, 1)
    wb38.wait()
    wb39.wait()


@functools.cache
def _scatter_sc_build():
    mesh = plsc.VectorSubcoreMesh(core_axis_name="c", subcore_axis_name="s",
                                  num_cores=NC, num_subcores=NS)
    return functools.partial(
        pl.kernel,
        out_type=jax.ShapeDtypeStruct((NC, NPAD, H), f32),
        mesh=mesh,
        scratch_types=[
            pltpu.VMEM((GB_FULL, GB_ROWS), jnp.int32),
            pltpu.VMEM((GB_TAIL,), jnp.int32),
            pltpu.VMEM((GB_ROWS, H), f32),
            pltpu.VMEM((GB_ROWS, H), f32),
            pltpu.VMEM_SHARED((NPAD, H), f32),
            pltpu.SemaphoreType.DMA,
            pltpu.SemaphoreType.DMA,
            pltpu.SemaphoreType.DMA,
        ],
    )(_scatter_sc_body)


def _scatter_sc(e_new, rmain, rtail, zeros):
    return _scatter_sc_build()(e_new, rmain, rtail, zeros)


def _scatter_sc_body(vals_hbm, rmain_hbm, rtail_hbm, zeros_hbm, out_hbm,
                     idxm, idxt, v0, v1, acc, sem_s0, sem_s1, sem_a):
    cid = lax.axis_index("c")
    sid = lax.axis_index("s")
    wid = sid * NC + cid
    base = pl.multiple_of(wid * EPW, 8)
    pltpu.sync_copy(rmain_hbm.at[wid], idxm)
    pltpu.sync_copy(rtail_hbm.at[wid], idxt)
    stripe = sid * STRIPE
    pltpu.sync_copy(zeros_hbm, acc.at[pl.ds(stripe, STRIPE)])
    plsc.subcore_barrier()

    def stage(w, n, vbuf, sem):
        voff = pl.multiple_of(base + w * GB_ROWS, 8)
        return pltpu.async_copy(vals_hbm.at[pl.ds(voff, n)],
                                vbuf.at[pl.ds(0, n)], sem)

    def pair(k, carry):
        w = 2 * k
        st0 = stage(w, GB_ROWS, v0, sem_s0)
        st1 = stage(w + 1, GB_ROWS, v1, sem_s1)
        st0.wait()
        h0 = pltpu.async_copy(v0, acc.at[idxm.at[w]], sem_a, add=True)
        st1.wait()
        h1 = pltpu.async_copy(v1, acc.at[idxm.at[w + 1]], sem_a, add=True)
        h0.wait()
        h1.wait()
        return carry

    lax.fori_loop(0, GB_FULL // 2, pair, 0)
    st0 = stage(GB_FULL - 1, GB_ROWS, v0, sem_s0)
    st1 = stage(GB_FULL, GB_TAIL, v1, sem_s1)
    st0.wait()
    h0 = pltpu.async_copy(v0, acc.at[idxm.at[GB_FULL - 1]], sem_a, add=True)
    st1.wait()
    h1 = pltpu.async_copy(v1.at[pl.ds(0, GB_TAIL)], acc.at[idxt], sem_a,
                          add=True)
    h0.wait()
    h1.wait()
    plsc.subcore_barrier()
    pltpu.sync_copy(acc.at[pl.ds(stripe, STRIPE)],
                    out_hbm.at[cid, pl.ds(stripe, STRIPE)])


def kernel(node_attr, edge_attr, edge_index, params):
    s = edge_index[0].astype(jnp.int32)
    r = edge_index[1].astype(jnp.int32)
    rr = r.reshape(NW, EPW)
    rmain = rr[:, :GB_FULL * GB_ROWS].reshape(NW, GB_FULL, GB_ROWS)
    rtail = rr[:, GB_FULL * GB_ROWS:]
    zeros = jnp.zeros((STRIPE, H), f32)

    nf = _encoder(node_attr, params['node_encoder'], NB)
    ef = _encoder(edge_attr, params['edge_encoder'], EB)

    for blk in params['blocks']:
        pE, pN = blk['edge_mlp'], blk['node_mlp']
        w0 = pE['W'][0].astype(bf16)
        a_tab, b_tab = _ab(nf, w0[:H], w0[H:2 * H])
        gsum = _gather_sc(a_tab, b_tab, s, r)
        e_new = _edge_mlp(gsum, ef, pE)
        parts = _scatter_sc(e_new, rmain, rtail, zeros)
        nf = _node_mlp(nf, parts[0, :N], parts[1, :N], pN)
        ef = e_new
    return _decoder(nf, params['decoder'], NB)


# speedup vs baseline: 1.5749x; 1.1925x over previous
import functools

import jax
import jax.numpy as jnp
from jax import lax
from jax.experimental import pallas as pl
from jax.experimental.pallas import tpu as pltpu
from jax.experimental.pallas import tpu_sc as plsc

N = 10000
E = 160000
H = 128
NC, NS = 2, 16
NW = NC * NS
EPW = E // NW
BATCH = 40
WAVE = 5
ROWS = BATCH * WAVE
NWAVES = EPW // ROWS
NCHUNK = EPW // BATCH
STRIPE = 640
NPAD = NS * STRIPE

f32 = jnp.float32
bf16 = jnp.bfloat16

EB = 8000
NB = 2000


def _dot(x, w):
    return jnp.dot(x.astype(bf16), w, preferred_element_type=f32)


def _tail(pre, w1, b1, w2, b2, w3, b3, g, beta):
    h = jnp.maximum(pre, 0.0)
    h = jnp.maximum(_dot(h, w1[...]) + b1[...], 0.0)
    h = jnp.maximum(_dot(h, w2[...]) + b2[...], 0.0)
    h = _dot(h, w3[...]) + b3[...]
    if g is not None:
        mu = jnp.mean(h, axis=-1, keepdims=True)
        var = jnp.mean((h - mu) ** 2, axis=-1, keepdims=True)
        h = (h - mu) * lax.rsqrt(var + 1e-5) * g[...] + beta[...]
    return h


def _full(shape):
    return pl.BlockSpec(shape, lambda i: (0,) * len(shape))


def _rows(block, width):
    return pl.BlockSpec((block, width), lambda i: (i, 0))


def _prep(p, lay_norm):
    ws = [w.astype(bf16) for w in p['W']]
    bs = [b.reshape(1, -1) for b in p['b']]
    if lay_norm:
        return ws, bs, p['g'].reshape(1, -1), p['beta'].reshape(1, -1)
    return ws, bs, None, None


def _enc_body(x, w0, b0, w1, b1, w2, b2, w3, b3, g, beta, out):
    pre = _dot(x[...], w0[...]) + b0[...]
    out[...] = _tail(pre, w1, b1, w2, b2, w3, b3, g, beta)


def _encoder(x, p, block):
    ws, bs, g, beta = _prep(p, True)
    rows, width = x.shape
    args = [x, ws[0], bs[0], ws[1], bs[1], ws[2], bs[2], ws[3], bs[3], g, beta]
    specs = [_rows(block, width)] + [_full(a.shape) for a in args[1:]]
    return pl.pallas_call(
        _enc_body,
        grid=(rows // block,),
        in_specs=specs,
        out_specs=_rows(block, H),
        out_shape=jax.ShapeDtypeStruct((rows, H), f32),
    )(*args)


def _dec_body(x, w0, b0, w1, b1, w2, b2, w3, b3, out):
    pre = _dot(x[...], w0[...]) + b0[...]
    out[...] = _tail(pre, w1, b1, w2, b2, w3, b3, None, None)


def _decoder(x, p, block):
    ws, bs, _, _ = _prep(p, False)
    rows = x.shape[0]
    out_w = p['W'][3].shape[1]
    args = [x, ws[0], bs[0], ws[1], bs[1], ws[2], bs[2], ws[3], bs[3]]
    specs = [_rows(block, H)] + [_full(a.shape) for a in args[1:]]
    return pl.pallas_call(
        _dec_body,
        grid=(rows // block,),
        in_specs=specs,
        out_specs=_rows(block, out_w),
        out_shape=jax.ShapeDtypeStruct((rows, out_w), f32),
    )(*args)


def _ab_body(nf, w0a, w0b, a_out, b_out):
    nfb = nf[...].astype(bf16)
    a_out[...] = jnp.dot(nfb, w0a[...], preferred_element_type=f32)
    b_out[...] = jnp.dot(nfb, w0b[...], preferred_element_type=f32)


def _ab(nf, w0a, w0b):
    args = [nf, w0a, w0b]
    specs = [_rows(NB, H), _full(w0a.shape), _full(w0b.shape)]
    return pl.pallas_call(
        _ab_body,
        grid=(N // NB,),
        in_specs=specs,
        out_specs=[_rows(NB, H), _rows(NB, H)],
        out_shape=[jax.ShapeDtypeStruct((N, H), f32),
                   jax.ShapeDtypeStruct((N, H), f32)],
    )(*args)


def _unpack_lo(x):
    return lax.bitcast_convert_type(x << 16, f32)


def _unpack_hi(x):
    return lax.bitcast_convert_type(x & jnp.int32(-65536), f32)


def _edge_body(gp, ef, w0c, b0, w1, b1, w2, b2, w3, b3, g, beta, out):
    efv = ef[...]
    gpv = gp[...]
    gsum = jnp.concatenate([_unpack_lo(gpv), _unpack_hi(gpv)], axis=-1)
    pre = gsum + _dot(efv, w0c[...]) + b0[...]
    out[...] = _tail(pre, w1, b1, w2, b2, w3, b3, g, beta) + efv


def _edge_mlp(gsum, ef, p):
    ws, bs, g, beta = _prep(p, True)
    k = jnp.arange(H // 2)
    lo = 32 * (k // 16) + k % 16
    perm = jnp.concatenate([lo, lo + 16])
    w0c = ws[0][2 * H:3 * H][:, perm]
    b0 = bs[0][:, perm]
    w1 = ws[1][perm, :]
    args = [gsum, ef, w0c, b0, w1, bs[1], ws[2], bs[2], ws[3], bs[3],
            g, beta]
    specs = ([_rows(EB, H // 2), _rows(EB, H)] +
             [_full(a.shape) for a in args[2:]])
    return pl.pallas_call(
        _edge_body,
        grid=(E // EB,),
        in_specs=specs,
        out_specs=_rows(EB, H),
        out_shape=jax.ShapeDtypeStruct((E, H), f32),
    )(*args)


def _node_body(nf, a0, a1, w0a, w0b, b0, w1, b1, w2, b2, w3, b3, g, beta,
               wa2, wb2, out, a_out, b_out):
    nfv = nf[...]
    agg = (a0[...] + a1[...]).astype(bf16)
    pre = (_dot(nfv, w0a[...]) +
           jnp.dot(agg, w0b[...], preferred_element_type=f32) + b0[...])
    nf_new = _tail(pre, w1, b1, w2, b2, w3, b3, g, beta) + nfv
    out[...] = nf_new
    nfb = nf_new.astype(bf16)
    a_out[...] = jnp.dot(nfb, wa2[...], preferred_element_type=f32)
    b_out[...] = jnp.dot(nfb, wb2[...], preferred_element_type=f32)


def _node_mlp(nf, a0, a1, p, w0_next):
    ws, bs, g, beta = _prep(p, True)
    w0a, w0b = ws[0][:H], ws[0][H:2 * H]
    args = [nf, a0, a1, w0a, w0b, bs[0], ws[1], bs[1], ws[2], bs[2], ws[3],
            bs[3], g, beta, w0_next[:H], w0_next[H:2 * H]]
    specs = [_rows(NB, H)] * 3 + [_full(a.shape) for a in args[3:]]
    return pl.pallas_call(
        _node_body,
        grid=(N // NB,),
        in_specs=specs,
        out_specs=[_rows(NB, H)] * 3,
        out_shape=[jax.ShapeDtypeStruct((N, H), f32),
                   jax.ShapeDtypeStruct((N, H), f32),
                   jax.ShapeDtypeStruct((N, H), f32)],
    )(*args)


GB_FULL = 39
GB_ROWS = 128
GB_TAIL = EPW - GB_FULL * GB_ROWS


@functools.cache
def _gather_sc_build():
    mesh = plsc.VectorSubcoreMesh(core_axis_name="c", subcore_axis_name="s",
                                  num_cores=NC, num_subcores=NS)
    return functools.partial(
        pl.kernel,
        out_type=jax.ShapeDtypeStruct((E, H // 2), jnp.int32),
        mesh=mesh,
        scratch_types=[
            pltpu.VMEM((EPW,), jnp.int32),
            pltpu.VMEM((EPW,), jnp.int32),
            pltpu.VMEM((GB_ROWS, H), f32),
            pltpu.VMEM((GB_ROWS, H), f32),
            pltpu.VMEM((GB_ROWS, H), f32),
            pltpu.VMEM((GB_ROWS, H), f32),
            pltpu.VMEM((GB_ROWS, H // 2), jnp.int32),
            pltpu.VMEM((GB_ROWS, H // 2), jnp.int32),
            pltpu.SemaphoreType.DMA,
            pltpu.SemaphoreType.DMA,
            pltpu.SemaphoreType.DMA,
            pltpu.SemaphoreType.DMA,
        ],
    )(_gather_sc_body)


def _gather_sc(a_tab, b_tab, s, r):
    return _gather_sc_build()(a_tab, b_tab, s, r)


def _gather_sc_body(a_hbm, b_hbm, s_hbm, r_hbm, g_hbm,
                    sidx, ridx, a0, b0, a1, b1, g0, g1,
                    sem_g0, sem_g1, sem_w0, sem_w1):
    cid = lax.axis_index("c")
    sid = lax.axis_index("s")
    wid = sid * NC + cid
    base = pl.multiple_of(wid * EPW, 8)
    pltpu.sync_copy(s_hbm.at[pl.ds(base, EPW)], sidx)
    pltpu.sync_copy(r_hbm.at[pl.ds(base, EPW)], ridx)

    abufs, bbufs, gbufs = (a0, a1), (b0, b1), (g0, g1)
    gsems, wsems = (sem_g0, sem_g1), (sem_w0, sem_w1)

    def fire(w, n, slot):
        o = pl.multiple_of(w * GB_ROWS, 8)
        pltpu.async_copy(a_hbm.at[sidx.at[pl.ds(o, n)]],
                         abufs[slot].at[pl.ds(0, n)], gsems[slot])
        pltpu.async_copy(b_hbm.at[ridx.at[pl.ds(o, n)]],
                         bbufs[slot].at[pl.ds(0, n)], gsems[slot])

    def wait_g(n, slot):
        pltpu.make_async_copy(a_hbm.at[pl.ds(0, n)],
                              abufs[slot].at[pl.ds(0, n)], gsems[slot]).wait()
        pltpu.make_async_copy(b_hbm.at[pl.ds(0, n)],
                              bbufs[slot].at[pl.ds(0, n)], gsems[slot]).wait()

    def wait_wb(slot):
        pltpu.make_async_copy(g_hbm.at[pl.ds(0, GB_ROWS)],
                              gbufs[slot], wsems[slot]).wait()

    def pack(n, slot):
        abuf, bbuf, gbuf = abufs[slot], bbufs[slot], gbufs[slot]

        def rnd(bits):
            return bits + 32767 + (lax.shift_right_logical(bits, 16) & 1)

        def row(i, c):
            for j in range(H // 32):
                lo = abuf[i, pl.ds(32 * j, 16)] + bbuf[i, pl.ds(32 * j, 16)]
                hi = (abuf[i, pl.ds(32 * j + 16, 16)] +
                      bbuf[i, pl.ds(32 * j + 16, 16)])
                lob = rnd(lax.bitcast_convert_type(lo, jnp.int32))
                hib = rnd(lax.bitcast_convert_type(hi, jnp.int32))
                gbuf[i, pl.ds(16 * j, 16)] = (
                    lax.shift_right_logical(lob, 16) |
                    (hib & jnp.int32(-65536)))
            return c
        lax.fori_loop(0, n, row, 0)

    def fire_wb(w, n, slot):
        off = pl.multiple_of(base + w * GB_ROWS, 8)
        return pltpu.async_copy(gbufs[slot].at[pl.ds(0, n)],
                                g_hbm.at[pl.ds(off, n)], wsems[slot])

    fire(0, GB_ROWS, 0)
    fire(1, GB_ROWS, 1)
    for w in (0, 1):
        wait_g(GB_ROWS, w)
        pack(GB_ROWS, w)
        fire_wb(w, GB_ROWS, w)
        fire(w + 2, GB_ROWS, w)

    def step(w, slot):
        wait_g(GB_ROWS, slot)
        wait_wb(slot)
        pack(GB_ROWS, slot)
        fire_wb(w, GB_ROWS, slot)
        fire(w + 2, GB_ROWS, slot)

    def body(k, carry):
        step(2 * k, 0)
        step(2 * k + 1, 1)
        return carry

    lax.fori_loop(1, 18, body, 0)
    step(36, 0)
    wait_g(GB_ROWS, 1)
    wait_wb(1)
    pack(GB_ROWS, 1)
    fire_wb(37, GB_ROWS, 1)
    fire(GB_FULL, GB_TAIL, 1)
    wait_g(GB_ROWS, 0)
    wait_wb(0)
    pack(GB_ROWS, 0)
    wb38 = fire_wb(38, GB_ROWS, 0)
    wait_g(GB_TAIL, 1)
    wait_wb(1)
    pack(GB_TAIL, 1)
    wb39 = fire_wb(GB_FULL, GB_TAIL, 1)
    wb38.wait()
    wb39.wait()


@functools.cache
def _scatter_sc_build():
    mesh = plsc.VectorSubcoreMesh(core_axis_name="c", subcore_axis_name="s",
                                  num_cores=NC, num_subcores=NS)
    return functools.partial(
        pl.kernel,
        out_type=jax.ShapeDtypeStruct((NC, NPAD, H), f32),
        mesh=mesh,
        scratch_types=[
            pltpu.VMEM((GB_FULL, GB_ROWS), jnp.int32),
            pltpu.VMEM((GB_TAIL,), jnp.int32),
            pltpu.VMEM((GB_ROWS, H), f32),
            pltpu.VMEM((GB_ROWS, H), f32),
            pltpu.VMEM_SHARED((NPAD, H), f32),
            pltpu.SemaphoreType.DMA,
            pltpu.SemaphoreType.DMA,
            pltpu.SemaphoreType.DMA,
        ],
    )(_scatter_sc_body)


def _scatter_sc(e_new, rmain, rtail, zeros):
    return _scatter_sc_build()(e_new, rmain, rtail, zeros)


def _scatter_sc_body(vals_hbm, rmain_hbm, rtail_hbm, zeros_hbm, out_hbm,
                     idxm, idxt, v0, v1, acc, sem_s0, sem_s1, sem_a):
    cid = lax.axis_index("c")
    sid = lax.axis_index("s")
    wid = sid * NC + cid
    base = pl.multiple_of(wid * EPW, 8)
    pltpu.sync_copy(rmain_hbm.at[wid], idxm)
    pltpu.sync_copy(rtail_hbm.at[wid], idxt)
    stripe = sid * STRIPE
    pltpu.sync_copy(zeros_hbm, acc.at[pl.ds(stripe, STRIPE)])
    plsc.subcore_barrier()

    def stage(w, n, vbuf, sem):
        voff = pl.multiple_of(base + w * GB_ROWS, 8)
        return pltpu.async_copy(vals_hbm.at[pl.ds(voff, n)],
                                vbuf.at[pl.ds(0, n)], sem)

    def pair(k, carry):
        w = 2 * k
        st0 = stage(w, GB_ROWS, v0, sem_s0)
        st1 = stage(w + 1, GB_ROWS, v1, sem_s1)
        st0.wait()
        h0 = pltpu.async_copy(v0, acc.at[idxm.at[w]], sem_a, add=True)
        st1.wait()
        h1 = pltpu.async_copy(v1, acc.at[idxm.at[w + 1]], sem_a, add=True)
        h0.wait()
        h1.wait()
        return carry

    lax.fori_loop(0, GB_FULL // 2, pair, 0)
    st0 = stage(GB_FULL - 1, GB_ROWS, v0, sem_s0)
    st1 = stage(GB_FULL, GB_TAIL, v1, sem_s1)
    st0.wait()
    h0 = pltpu.async_copy(v0, acc.at[idxm.at[GB_FULL - 1]], sem_a, add=True)
    st1.wait()
    h1 = pltpu.async_copy(v1.at[pl.ds(0, GB_TAIL)], acc.at[idxt], sem_a,
                          add=True)
    h0.wait()
    h1.wait()
    plsc.subcore_barrier()
    pltpu.sync_copy(acc.at[pl.ds(stripe, STRIPE)],
                    out_hbm.at[cid, pl.ds(stripe, STRIPE)])


def kernel(node_attr, edge_attr, edge_index, params):
    s = edge_index[0].astype(jnp.int32)
    r = edge_index[1].astype(jnp.int32)
    rr = r.reshape(NW, EPW)
    rmain = rr[:, :GB_FULL * GB_ROWS].reshape(NW, GB_FULL, GB_ROWS)
    rtail = rr[:, GB_FULL * GB_ROWS:]
    zeros = jnp.zeros((STRIPE, H), f32)

    nf = _encoder(node_attr, params['node_encoder'], NB)
    ef = _encoder(edge_attr, params['edge_encoder'], EB)

    blocks = params['blocks']
    w0s = [blk['edge_mlp']['W'][0].astype(bf16) for blk in blocks]
    a_tab, b_tab = _ab(nf, w0s[0][:H], w0s[0][H:2 * H])
    for i, blk in enumerate(blocks):
        pE, pN = blk['edge_mlp'], blk['node_mlp']
        gsum = _gather_sc(a_tab, b_tab, s, r)
        e_new = _edge_mlp(gsum, ef, pE)
        parts = _scatter_sc(e_new, rmain, rtail, zeros)
        w0_next = w0s[i + 1] if i + 1 < len(blocks) else w0s[0]
        nf, a_tab, b_tab = _node_mlp(nf, parts[0, :N], parts[1, :N], pN,
                                     w0_next)
        ef = e_new
    return _decoder(nf, params['decoder'], NB)


# speedup vs baseline: 1.5932x; 1.0116x over previous
import functools

import jax
import jax.numpy as jnp
from jax import lax
from jax.experimental import pallas as pl
from jax.experimental.pallas import tpu as pltpu
from jax.experimental.pallas import tpu_sc as plsc

N = 10000
E = 160000
H = 128
NC, NS = 2, 16
NW = NC * NS
EPW = E // NW
BATCH = 40
WAVE = 5
ROWS = BATCH * WAVE
NWAVES = EPW // ROWS
NCHUNK = EPW // BATCH
STRIPE = 640
NPAD = NS * STRIPE

f32 = jnp.float32
bf16 = jnp.bfloat16

EB = 16000
NB = 2000


def _dot(x, w):
    return jnp.dot(x.astype(bf16), w, preferred_element_type=f32)


def _tail(pre, w1, b1, w2, b2, w3, b3, g, beta):
    h = jnp.maximum(pre, 0.0)
    h = jnp.maximum(_dot(h, w1[...]) + b1[...], 0.0)
    h = jnp.maximum(_dot(h, w2[...]) + b2[...], 0.0)
    h = _dot(h, w3[...]) + b3[...]
    if g is not None:
        mu = jnp.mean(h, axis=-1, keepdims=True)
        var = jnp.mean((h - mu) ** 2, axis=-1, keepdims=True)
        h = (h - mu) * lax.rsqrt(var + 1e-5) * g[...] + beta[...]
    return h


def _full(shape):
    return pl.BlockSpec(shape, lambda i: (0,) * len(shape))


def _rows(block, width):
    return pl.BlockSpec((block, width), lambda i: (i, 0))


def _prep(p, lay_norm):
    ws = [w.astype(bf16) for w in p['W']]
    bs = [b.reshape(1, -1) for b in p['b']]
    if lay_norm:
        return ws, bs, p['g'].reshape(1, -1), p['beta'].reshape(1, -1)
    return ws, bs, None, None


def _enc_body(x, w0, b0, w1, b1, w2, b2, w3, b3, g, beta, out):
    pre = _dot(x[...], w0[...]) + b0[...]
    out[...] = _tail(pre, w1, b1, w2, b2, w3, b3, g, beta)


def _encoder(x, p, block):
    ws, bs, g, beta = _prep(p, True)
    rows, width = x.shape
    args = [x, ws[0], bs[0], ws[1], bs[1], ws[2], bs[2], ws[3], bs[3], g, beta]
    specs = [_rows(block, width)] + [_full(a.shape) for a in args[1:]]
    return pl.pallas_call(
        _enc_body,
        grid=(rows // block,),
        in_specs=specs,
        out_specs=_rows(block, H),
        out_shape=jax.ShapeDtypeStruct((rows, H), f32),
    )(*args)


def _dec_body(x, w0, b0, w1, b1, w2, b2, w3, b3, out):
    pre = _dot(x[...], w0[...]) + b0[...]
    out[...] = _tail(pre, w1, b1, w2, b2, w3, b3, None, None)


def _decoder(x, p, block):
    ws, bs, _, _ = _prep(p, False)
    rows = x.shape[0]
    out_w = p['W'][3].shape[1]
    args = [x, ws[0], bs[0], ws[1], bs[1], ws[2], bs[2], ws[3], bs[3]]
    specs = [_rows(block, H)] + [_full(a.shape) for a in args[1:]]
    return pl.pallas_call(
        _dec_body,
        grid=(rows // block,),
        in_specs=specs,
        out_specs=_rows(block, out_w),
        out_shape=jax.ShapeDtypeStruct((rows, out_w), f32),
    )(*args)


def _ab_body(nf, w0a, w0b, a_out, b_out):
    nfb = nf[...].astype(bf16)
    a_out[...] = jnp.dot(nfb, w0a[...], preferred_element_type=f32)
    b_out[...] = jnp.dot(nfb, w0b[...], preferred_element_type=f32)


def _ab(nf, w0a, w0b):
    args = [nf, w0a, w0b]
    specs = [_rows(NB, H), _full(w0a.shape), _full(w0b.shape)]
    return pl.pallas_call(
        _ab_body,
        grid=(N // NB,),
        in_specs=specs,
        out_specs=[_rows(NB, H), _rows(NB, H)],
        out_shape=[jax.ShapeDtypeStruct((N, H), f32),
                   jax.ShapeDtypeStruct((N, H), f32)],
    )(*args)


def _unpack_lo(x):
    return lax.bitcast_convert_type(x << 16, f32)


def _unpack_hi(x):
    return lax.bitcast_convert_type(x & jnp.int32(-65536), f32)


def _edge_body(gp, ef, w0c, b0, w1, b1, w2, b2, w3, b3, g, beta, out):
    efv = ef[...]
    gpv = gp[...]
    gsum = jnp.concatenate([_unpack_lo(gpv), _unpack_hi(gpv)], axis=-1)
    pre = gsum + _dot(efv, w0c[...]) + b0[...]
    out[...] = _tail(pre, w1, b1, w2, b2, w3, b3, g, beta) + efv


def _edge_mlp(gsum, ef, p):
    ws, bs, g, beta = _prep(p, True)
    k = jnp.arange(H // 2)
    lo = 32 * (k // 16) + k % 16
    perm = jnp.concatenate([lo, lo + 16])
    w0c = ws[0][2 * H:3 * H][:, perm]
    b0 = bs[0][:, perm]
    w1 = ws[1][perm, :]
    args = [gsum, ef, w0c, b0, w1, bs[1], ws[2], bs[2], ws[3], bs[3],
            g, beta]
    specs = ([_rows(EB, H // 2), _rows(EB, H)] +
             [_full(a.shape) for a in args[2:]])
    return pl.pallas_call(
        _edge_body,
        grid=(E // EB,),
        in_specs=specs,
        out_specs=_rows(EB, H),
        out_shape=jax.ShapeDtypeStruct((E, H), f32),
    )(*args)


def _node_body(nf, a0, a1, w0a, w0b, b0, w1, b1, w2, b2, w3, b3, g, beta,
               wa2, wb2, out, a_out, b_out):
    nfv = nf[...]
    agg = (a0[...] + a1[...]).astype(bf16)
    pre = (_dot(nfv, w0a[...]) +
           jnp.dot(agg, w0b[...], preferred_element_type=f32) + b0[...])
    nf_new = _tail(pre, w1, b1, w2, b2, w3, b3, g, beta) + nfv
    out[...] = nf_new
    nfb = nf_new.astype(bf16)
    a_out[...] = jnp.dot(nfb, wa2[...], preferred_element_type=f32)
    b_out[...] = jnp.dot(nfb, wb2[...], preferred_element_type=f32)


def _node_mlp(nf, a0, a1, p, w0_next):
    ws, bs, g, beta = _prep(p, True)
    w0a, w0b = ws[0][:H], ws[0][H:2 * H]
    args = [nf, a0, a1, w0a, w0b, bs[0], ws[1], bs[1], ws[2], bs[2], ws[3],
            bs[3], g, beta, w0_next[:H], w0_next[H:2 * H]]
    specs = [_rows(NB, H)] * 3 + [_full(a.shape) for a in args[3:]]
    return pl.pallas_call(
        _node_body,
        grid=(N // NB,),
        in_specs=specs,
        out_specs=[_rows(NB, H)] * 3,
        out_shape=[jax.ShapeDtypeStruct((N, H), f32),
                   jax.ShapeDtypeStruct((N, H), f32),
                   jax.ShapeDtypeStruct((N, H), f32)],
    )(*args)


GB_FULL = 39
GB_ROWS = 128
GB_TAIL = EPW - GB_FULL * GB_ROWS


@functools.cache
def _gather_sc_build():
    mesh = plsc.VectorSubcoreMesh(core_axis_name="c", subcore_axis_name="s",
                                  num_cores=NC, num_subcores=NS)
    return functools.partial(
        pl.kernel,
        out_type=jax.ShapeDtypeStruct((E, H // 2), jnp.int32),
        mesh=mesh,
        scratch_types=[
            pltpu.VMEM((EPW,), jnp.int32),
            pltpu.VMEM((EPW,), jnp.int32),
            pltpu.VMEM((GB_ROWS, H), f32),
            pltpu.VMEM((GB_ROWS, H), f32),
            pltpu.VMEM((GB_ROWS, H), f32),
            pltpu.VMEM((GB_ROWS, H), f32),
            pltpu.VMEM((GB_ROWS, H // 2), jnp.int32),
            pltpu.VMEM((GB_ROWS, H // 2), jnp.int32),
            pltpu.SemaphoreType.DMA,
            pltpu.SemaphoreType.DMA,
            pltpu.SemaphoreType.DMA,
            pltpu.SemaphoreType.DMA,
        ],
    )(_gather_sc_body)


def _gather_sc(a_tab, b_tab, s, r):
    return _gather_sc_build()(a_tab, b_tab, s, r)


def _gather_sc_body(a_hbm, b_hbm, s_hbm, r_hbm, g_hbm,
                    sidx, ridx, a0, b0, a1, b1, g0, g1,
                    sem_g0, sem_g1, sem_w0, sem_w1):
    cid = lax.axis_index("c")
    sid = lax.axis_index("s")
    wid = sid * NC + cid
    base = pl.multiple_of(wid * EPW, 8)
    pltpu.sync_copy(s_hbm.at[pl.ds(base, EPW)], sidx)
    pltpu.sync_copy(r_hbm.at[pl.ds(base, EPW)], ridx)

    abufs, bbufs, gbufs = (a0, a1), (b0, b1), (g0, g1)
    gsems, wsems = (sem_g0, sem_g1), (sem_w0, sem_w1)

    def fire(w, n, slot):
        o = pl.multiple_of(w * GB_ROWS, 8)
        pltpu.async_copy(a_hbm.at[sidx.at[pl.ds(o, n)]],
                         abufs[slot].at[pl.ds(0, n)], gsems[slot])
        pltpu.async_copy(b_hbm.at[ridx.at[pl.ds(o, n)]],
                         bbufs[slot].at[pl.ds(0, n)], gsems[slot])

    def wait_g(n, slot):
        pltpu.make_async_copy(a_hbm.at[pl.ds(0, n)],
                              abufs[slot].at[pl.ds(0, n)], gsems[slot]).wait()
        pltpu.make_async_copy(b_hbm.at[pl.ds(0, n)],
                              bbufs[slot].at[pl.ds(0, n)], gsems[slot]).wait()

    def wait_wb(slot):
        pltpu.make_async_copy(g_hbm.at[pl.ds(0, GB_ROWS)],
                              gbufs[slot], wsems[slot]).wait()

    def pack(n, slot):
        abuf, bbuf, gbuf = abufs[slot], bbufs[slot], gbufs[slot]

        def rnd(bits):
            return bits + 32767 + (lax.shift_right_logical(bits, 16) & 1)

        def row(i, c):
            for j in range(H // 32):
                lo = abuf[i, pl.ds(32 * j, 16)] + bbuf[i, pl.ds(32 * j, 16)]
                hi = (abuf[i, pl.ds(32 * j + 16, 16)] +
                      bbuf[i, pl.ds(32 * j + 16, 16)])
                lob = rnd(lax.bitcast_convert_type(lo, jnp.int32))
                hib = rnd(lax.bitcast_convert_type(hi, jnp.int32))
                gbuf[i, pl.ds(16 * j, 16)] = (
                    lax.shift_right_logical(lob, 16) |
                    (hib & jnp.int32(-65536)))
            return c
        lax.fori_loop(0, n, row, 0)

    def fire_wb(w, n, slot):
        off = pl.multiple_of(base + w * GB_ROWS, 8)
        return pltpu.async_copy(gbufs[slot].at[pl.ds(0, n)],
                                g_hbm.at[pl.ds(off, n)], wsems[slot])

    fire(0, GB_ROWS, 0)
    fire(1, GB_ROWS, 1)
    for w in (0, 1):
        wait_g(GB_ROWS, w)
        pack(GB_ROWS, w)
        fire_wb(w, GB_ROWS, w)
        fire(w + 2, GB_ROWS, w)

    def step(w, slot):
        wait_g(GB_ROWS, slot)
        wait_wb(slot)
        pack(GB_ROWS, slot)
        fire_wb(w, GB_ROWS, slot)
        fire(w + 2, GB_ROWS, slot)

    def body(k, carry):
        step(2 * k, 0)
        step(2 * k + 1, 1)
        return carry

    lax.fori_loop(1, 18, body, 0)
    step(36, 0)
    wait_g(GB_ROWS, 1)
    wait_wb(1)
    pack(GB_ROWS, 1)
    fire_wb(37, GB_ROWS, 1)
    fire(GB_FULL, GB_TAIL, 1)
    wait_g(GB_ROWS, 0)
    wait_wb(0)
    pack(GB_ROWS, 0)
    wb38 = fire_wb(38, GB_ROWS, 0)
    wait_g(GB_TAIL, 1)
    wait_wb(1)
    pack(GB_TAIL, 1)
    wb39 = fire_wb(GB_FULL, GB_TAIL, 1)
    wb38.wait()
    wb39.wait()


@functools.cache
def _scatter_sc_build():
    mesh = plsc.VectorSubcoreMesh(core_axis_name="c", subcore_axis_name="s",
                                  num_cores=NC, num_subcores=NS)
    return functools.partial(
        pl.kernel,
        out_type=jax.ShapeDtypeStruct((NC, NPAD, H), f32),
        mesh=mesh,
        scratch_types=[
            pltpu.VMEM((GB_FULL, GB_ROWS), jnp.int32),
            pltpu.VMEM((GB_TAIL,), jnp.int32),
            pltpu.VMEM((GB_ROWS, H), f32),
            pltpu.VMEM((GB_ROWS, H), f32),
            pltpu.VMEM_SHARED((NPAD, H), f32),
            pltpu.SemaphoreType.DMA,
            pltpu.SemaphoreType.DMA,
            pltpu.SemaphoreType.DMA,
        ],
    )(_scatter_sc_body)


def _scatter_sc(e_new, rmain, rtail, zeros):
    return _scatter_sc_build()(e_new, rmain, rtail, zeros)


def _scatter_sc_body(vals_hbm, rmain_hbm, rtail_hbm, zeros_hbm, out_hbm,
                     idxm, idxt, v0, v1, acc, sem_s0, sem_s1, sem_a):
    cid = lax.axis_index("c")
    sid = lax.axis_index("s")
    wid = sid * NC + cid
    base = pl.multiple_of(wid * EPW, 8)
    pltpu.sync_copy(rmain_hbm.at[wid], idxm)
    pltpu.sync_copy(rtail_hbm.at[wid], idxt)
    stripe = sid * STRIPE
    pltpu.sync_copy(zeros_hbm, acc.at[pl.ds(stripe, STRIPE)])
    plsc.subcore_barrier()

    def stage(w, n, vbuf, sem):
        voff = pl.multiple_of(base + w * GB_ROWS, 8)
        return pltpu.async_copy(vals_hbm.at[pl.ds(voff, n)],
                                vbuf.at[pl.ds(0, n)], sem)

    def pair(k, carry):
        w = 2 * k
        st0 = stage(w, GB_ROWS, v0, sem_s0)
        st1 = stage(w + 1, GB_ROWS, v1, sem_s1)
        st0.wait()
        h0 = pltpu.async_copy(v0, acc.at[idxm.at[w]], sem_a, add=True)
        st1.wait()
        h1 = pltpu.async_copy(v1, acc.at[idxm.at[w + 1]], sem_a, add=True)
        h0.wait()
        h1.wait()
        return carry

    lax.fori_loop(0, GB_FULL // 2, pair, 0)
    st0 = stage(GB_FULL - 1, GB_ROWS, v0, sem_s0)
    st1 = stage(GB_FULL, GB_TAIL, v1, sem_s1)
    st0.wait()
    h0 = pltpu.async_copy(v0, acc.at[idxm.at[GB_FULL - 1]], sem_a, add=True)
    st1.wait()
    h1 = pltpu.async_copy(v1.at[pl.ds(0, GB_TAIL)], acc.at[idxt], sem_a,
                          add=True)
    h0.wait()
    h1.wait()
    plsc.subcore_barrier()
    pltpu.sync_copy(acc.at[pl.ds(stripe, STRIPE)],
                    out_hbm.at[cid, pl.ds(stripe, STRIPE)])


def kernel(node_attr, edge_attr, edge_index, params):
    s = edge_index[0].astype(jnp.int32)
    r = edge_index[1].astype(jnp.int32)
    rr = r.reshape(NW, EPW)
    rmain = rr[:, :GB_FULL * GB_ROWS].reshape(NW, GB_FULL, GB_ROWS)
    rtail = rr[:, GB_FULL * GB_ROWS:]
    zeros = jnp.zeros((STRIPE, H), f32)

    nf = _encoder(node_attr, params['node_encoder'], NB)
    ef = _encoder(edge_attr, params['edge_encoder'], EB)

    blocks = params['blocks']
    w0s = [blk['edge_mlp']['W'][0].astype(bf16) for blk in blocks]
    a_tab, b_tab = _ab(nf, w0s[0][:H], w0s[0][H:2 * H])
    for i, blk in enumerate(blocks):
        pE, pN = blk['edge_mlp'], blk['node_mlp']
        gsum = _gather_sc(a_tab, b_tab, s, r)
        e_new = _edge_mlp(gsum, ef, pE)
        parts = _scatter_sc(e_new, rmain, rtail, zeros)
        w0_next = w0s[i + 1] if i + 1 < len(blocks) else w0s[0]
        nf, a_tab, b_tab = _node_mlp(nf, parts[0, :N], parts[1, :N], pN,
                                     w0_next)
        ef = e_new
    return _decoder(nf, params['decoder'], NB)


# speedup vs baseline: 1.6049x; 1.0073x over previous
import functools

import jax
import jax.numpy as jnp
from jax import lax
from jax.experimental import pallas as pl
from jax.experimental.pallas import tpu as pltpu
from jax.experimental.pallas import tpu_sc as plsc

N = 10000
E = 160000
H = 128
NC, NS = 2, 16
NW = NC * NS
EPW = E // NW
BATCH = 40
WAVE = 5
ROWS = BATCH * WAVE
NWAVES = EPW // ROWS
NCHUNK = EPW // BATCH
STRIPE = 640
NPAD = NS * STRIPE

f32 = jnp.float32
bf16 = jnp.bfloat16

EB = 8000
NB = 2000


def _dot(x, w):
    return jnp.dot(x.astype(bf16), w.astype(bf16),
                   preferred_element_type=f32)


def _dot32(x, w):
    return jnp.dot(x, w, preferred_element_type=f32)


def _tail(pre, w1, b1, w2, b2, w3, b3, g, beta, dot=_dot):
    h = jnp.maximum(pre, 0.0)
    h = jnp.maximum(dot(h, w1[...]) + b1[...], 0.0)
    h = jnp.maximum(dot(h, w2[...]) + b2[...], 0.0)
    h = dot(h, w3[...]) + b3[...]
    if g is not None:
        mu = jnp.mean(h, axis=-1, keepdims=True)
        var = jnp.mean((h - mu) ** 2, axis=-1, keepdims=True)
        h = (h - mu) * lax.rsqrt(var + 1e-5) * g[...] + beta[...]
    return h


def _full(shape):
    return pl.BlockSpec(shape, lambda i: (0,) * len(shape))


def _rows(block, width):
    return pl.BlockSpec((block, width), lambda i: (i, 0))


def _prep(p, lay_norm):
    ws = list(p['W'])
    bs = [b.reshape(1, -1) for b in p['b']]
    if lay_norm:
        return ws, bs, p['g'].reshape(1, -1), p['beta'].reshape(1, -1)
    return ws, bs, None, None


def _enc_body(x, w0, b0, w1, b1, w2, b2, w3, b3, g, beta, out):
    pre = _dot32(x[...], w0[...]) + b0[...]
    out[...] = _tail(pre, w1, b1, w2, b2, w3, b3, g, beta, dot=_dot32)


def _encoder(x, p, block):
    ws, bs, g, beta = _prep(p, True)
    rows, width = x.shape
    args = [x, ws[0], bs[0], ws[1], bs[1], ws[2], bs[2], ws[3], bs[3], g, beta]
    specs = [_rows(block, width)] + [_full(a.shape) for a in args[1:]]
    return pl.pallas_call(
        _enc_body,
        grid=(rows // block,),
        in_specs=specs,
        out_specs=_rows(block, H),
        out_shape=jax.ShapeDtypeStruct((rows, H), f32),
    )(*args)


def _dec_body(x, w0, b0, w1, b1, w2, b2, w3, b3, out):
    pre = _dot32(x[...], w0[...]) + b0[...]
    out[...] = _tail(pre, w1, b1, w2, b2, w3, b3, None, None, dot=_dot32)


def _decoder(x, p, block):
    ws, bs, _, _ = _prep(p, False)
    rows = x.shape[0]
    out_w = p['W'][3].shape[1]
    args = [x, ws[0], bs[0], ws[1], bs[1], ws[2], bs[2], ws[3], bs[3]]
    specs = [_rows(block, H)] + [_full(a.shape) for a in args[1:]]
    return pl.pallas_call(
        _dec_body,
        grid=(rows // block,),
        in_specs=specs,
        out_specs=_rows(block, out_w),
        out_shape=jax.ShapeDtypeStruct((rows, out_w), f32),
    )(*args)


def _ab_body(nf, w0a, w0b, a_out, b_out):
    nfv = nf[...]
    a_out[...] = _dot32(nfv, w0a[...])
    b_out[...] = _dot32(nfv, w0b[...])


def _ab(nf, w0a, w0b):
    args = [nf, w0a, w0b]
    specs = [_rows(NB, H), _full(w0a.shape), _full(w0b.shape)]
    return pl.pallas_call(
        _ab_body,
        grid=(N // NB,),
        in_specs=specs,
        out_specs=[_rows(NB, H), _rows(NB, H)],
        out_shape=[jax.ShapeDtypeStruct((N, H), f32),
                   jax.ShapeDtypeStruct((N, H), f32)],
    )(*args)


def _unpack_lo(x):
    return lax.bitcast_convert_type(x << 16, f32)


def _unpack_hi(x):
    return lax.bitcast_convert_type(x & jnp.int32(-65536), f32)


def _edge_body(gp, ef, w0c, b0, w1, b1, w2, b2, w3, b3, g, beta, out):
    efv = ef[...]
    pre = gp[...] + _dot32(efv, w0c[...]) + b0[...]
    out[...] = _tail(pre, w1, b1, w2, b2, w3, b3, g, beta, dot=_dot32) + efv


def _edge_mlp(gsum, ef, p):
    ws, bs, g, beta = _prep(p, True)
    w0c = ws[0][2 * H:3 * H]
    args = [gsum, ef, w0c, bs[0], ws[1], bs[1], ws[2], bs[2], ws[3], bs[3],
            g, beta]
    specs = ([_rows(EB, H), _rows(EB, H)] +
             [_full(a.shape) for a in args[2:]])
    return pl.pallas_call(
        _edge_body,
        grid=(E // EB,),
        in_specs=specs,
        out_specs=_rows(EB, H),
        out_shape=jax.ShapeDtypeStruct((E, H), f32),
    )(*args)


def _node_body(nf, a0, a1, w0a, w0b, b0, w1, b1, w2, b2, w3, b3, g, beta,
               wa2, wb2, out, a_out, b_out):
    nfv = nf[...]
    agg = a0[...] + a1[...]
    pre = _dot32(nfv, w0a[...]) + _dot32(agg, w0b[...]) + b0[...]
    nf_new = _tail(pre, w1, b1, w2, b2, w3, b3, g, beta, dot=_dot32) + nfv
    out[...] = nf_new
    a_out[...] = _dot32(nf_new, wa2[...])
    b_out[...] = _dot32(nf_new, wb2[...])


def _node_mlp(nf, a0, a1, p, w0_next):
    ws, bs, g, beta = _prep(p, True)
    w0a, w0b = ws[0][:H], ws[0][H:2 * H]
    args = [nf, a0, a1, w0a, w0b, bs[0], ws[1], bs[1], ws[2], bs[2], ws[3],
            bs[3], g, beta, w0_next[:H], w0_next[H:2 * H]]
    specs = [_rows(NB, H)] * 3 + [_full(a.shape) for a in args[3:]]
    return pl.pallas_call(
        _node_body,
        grid=(N // NB,),
        in_specs=specs,
        out_specs=[_rows(NB, H)] * 3,
        out_shape=[jax.ShapeDtypeStruct((N, H), f32),
                   jax.ShapeDtypeStruct((N, H), f32),
                   jax.ShapeDtypeStruct((N, H), f32)],
    )(*args)


GB_FULL = 39
GB_ROWS = 128
GB_TAIL = EPW - GB_FULL * GB_ROWS


@functools.cache
def _gather_sc_build():
    mesh = plsc.VectorSubcoreMesh(core_axis_name="c", subcore_axis_name="s",
                                  num_cores=NC, num_subcores=NS)
    return functools.partial(
        pl.kernel,
        out_type=jax.ShapeDtypeStruct((E, H), f32),
        mesh=mesh,
        scratch_types=[
            pltpu.VMEM((EPW,), jnp.int32),
            pltpu.VMEM((EPW,), jnp.int32),
            pltpu.VMEM((GB_ROWS, H), f32),
            pltpu.VMEM((GB_ROWS, H), f32),
            pltpu.VMEM((GB_ROWS, H), f32),
            pltpu.VMEM((GB_ROWS, H), f32),
            pltpu.VMEM((GB_ROWS, H), f32),
            pltpu.VMEM((GB_ROWS, H), f32),
            pltpu.SemaphoreType.DMA,
            pltpu.SemaphoreType.DMA,
            pltpu.SemaphoreType.DMA,
            pltpu.SemaphoreType.DMA,
        ],
    )(_gather_sc_body)


def _gather_sc(a_tab, b_tab, s, r):
    return _gather_sc_build()(a_tab, b_tab, s, r)


def _gather_sc_body(a_hbm, b_hbm, s_hbm, r_hbm, g_hbm,
                    sidx, ridx, a0, b0, a1, b1, g0, g1,
                    sem_g0, sem_g1, sem_w0, sem_w1):
    cid = lax.axis_index("c")
    sid = lax.axis_index("s")
    wid = sid * NC + cid
    base = pl.multiple_of(wid * EPW, 8)
    pltpu.sync_copy(s_hbm.at[pl.ds(base, EPW)], sidx)
    pltpu.sync_copy(r_hbm.at[pl.ds(base, EPW)], ridx)

    abufs, bbufs, gbufs = (a0, a1), (b0, b1), (g0, g1)
    gsems, wsems = (sem_g0, sem_g1), (sem_w0, sem_w1)

    def fire(w, n, slot):
        o = pl.multiple_of(w * GB_ROWS, 8)
        pltpu.async_copy(a_hbm.at[sidx.at[pl.ds(o, n)]],
                         abufs[slot].at[pl.ds(0, n)], gsems[slot])
        pltpu.async_copy(b_hbm.at[ridx.at[pl.ds(o, n)]],
                         bbufs[slot].at[pl.ds(0, n)], gsems[slot])

    def wait_g(n, slot):
        pltpu.make_async_copy(a_hbm.at[pl.ds(0, n)],
                              abufs[slot].at[pl.ds(0, n)], gsems[slot]).wait()
        pltpu.make_async_copy(b_hbm.at[pl.ds(0, n)],
                              bbufs[slot].at[pl.ds(0, n)], gsems[slot]).wait()

    def wait_wb(slot):
        pltpu.make_async_copy(g_hbm.at[pl.ds(0, GB_ROWS)],
                              gbufs[slot], wsems[slot]).wait()

    def pack(n, slot):
        abuf, bbuf, gbuf = abufs[slot], bbufs[slot], gbufs[slot]

        def row(i, c):
            for j in range(H // 16):
                sl = pl.ds(j * 16, 16)
                gbuf[i, sl] = abuf[i, sl] + bbuf[i, sl]
            return c
        lax.fori_loop(0, n, row, 0)

    def fire_wb(w, n, slot):
        off = pl.multiple_of(base + w * GB_ROWS, 8)
        return pltpu.async_copy(gbufs[slot].at[pl.ds(0, n)],
                                g_hbm.at[pl.ds(off, n)], wsems[slot])

    fire(0, GB_ROWS, 0)
    fire(1, GB_ROWS, 1)
    for w in (0, 1):
        wait_g(GB_ROWS, w)
        pack(GB_ROWS, w)
        fire_wb(w, GB_ROWS, w)
        fire(w + 2, GB_ROWS, w)

    def step(w, slot):
        wait_g(GB_ROWS, slot)
        wait_wb(slot)
        pack(GB_ROWS, slot)
        fire_wb(w, GB_ROWS, slot)
        fire(w + 2, GB_ROWS, slot)

    def body(k, carry):
        step(2 * k, 0)
        step(2 * k + 1, 1)
        return carry

    lax.fori_loop(1, 18, body, 0)
    step(36, 0)
    wait_g(GB_ROWS, 1)
    wait_wb(1)
    pack(GB_ROWS, 1)
    fire_wb(37, GB_ROWS, 1)
    fire(GB_FULL, GB_TAIL, 1)
    wait_g(GB_ROWS, 0)
    wait_wb(0)
    pack(GB_ROWS, 0)
    wb38 = fire_wb(38, GB_ROWS, 0)
    wait_g(GB_TAIL, 1)
    wait_wb(1)
    pack(GB_TAIL, 1)
    wb39 = fire_wb(GB_FULL, GB_TAIL, 1)
    wb38.wait()
    wb39.wait()


@functools.cache
def _scatter_sc_build():
    mesh = plsc.VectorSubcoreMesh(core_axis_name="c", subcore_axis_name="s",
                                  num_cores=NC, num_subcores=NS)
    return functools.partial(
        pl.kernel,
        out_type=jax.ShapeDtypeStruct((NC, NPAD, H), f32),
        mesh=mesh,
        scratch_types=[
            pltpu.VMEM((GB_FULL, GB_ROWS), jnp.int32),
            pltpu.VMEM((GB_TAIL,), jnp.int32),
            pltpu.VMEM((GB_ROWS, H), f32),
            pltpu.VMEM((GB_ROWS, H), f32),
            pltpu.VMEM_SHARED((NPAD, H), f32),
            pltpu.SemaphoreType.DMA,
            pltpu.SemaphoreType.DMA,
            pltpu.SemaphoreType.DMA,
        ],
    )(_scatter_sc_body)


def _scatter_sc(e_new, rmain, rtail, zeros):
    return _scatter_sc_build()(e_new, rmain, rtail, zeros)


def _scatter_sc_body(vals_hbm, rmain_hbm, rtail_hbm, zeros_hbm, out_hbm,
                     idxm, idxt, v0, v1, acc, sem_s0, sem_s1, sem_a):
    cid = lax.axis_index("c")
    sid = lax.axis_index("s")
    wid = sid * NC + cid
    base = pl.multiple_of(wid * EPW, 8)
    pltpu.sync_copy(rmain_hbm.at[wid], idxm)
    pltpu.sync_copy(rtail_hbm.at[wid], idxt)
    stripe = sid * STRIPE
    pltpu.sync_copy(zeros_hbm, acc.at[pl.ds(stripe, STRIPE)])
    plsc.subcore_barrier()

    def stage(w, n, vbuf, sem):
        voff = pl.multiple_of(base + w * GB_ROWS, 8)
        return pltpu.async_copy(vals_hbm.at[pl.ds(voff, n)],
                                vbuf.at[pl.ds(0, n)], sem)

    def pair(k, carry):
        w = 2 * k
        st0 = stage(w, GB_ROWS, v0, sem_s0)
        st1 = stage(w + 1, GB_ROWS, v1, sem_s1)
        st0.wait()
        h0 = pltpu.async_copy(v0, acc.at[idxm.at[w]], sem_a, add=True)
        st1.wait()
        h1 = pltpu.async_copy(v1, acc.at[idxm.at[w + 1]], sem_a, add=True)
        h0.wait()
        h1.wait()
        return carry

    lax.fori_loop(0, GB_FULL // 2, pair, 0)
    st0 = stage(GB_FULL - 1, GB_ROWS, v0, sem_s0)
    st1 = stage(GB_FULL, GB_TAIL, v1, sem_s1)
    st0.wait()
    h0 = pltpu.async_copy(v0, acc.at[idxm.at[GB_FULL - 1]], sem_a, add=True)
    st1.wait()
    h1 = pltpu.async_copy(v1.at[pl.ds(0, GB_TAIL)], acc.at[idxt], sem_a,
                          add=True)
    h0.wait()
    h1.wait()
    plsc.subcore_barrier()
    pltpu.sync_copy(acc.at[pl.ds(stripe, STRIPE)],
                    out_hbm.at[cid, pl.ds(stripe, STRIPE)])


def kernel(node_attr, edge_attr, edge_index, params):
    s = edge_index[0].astype(jnp.int32)
    r = edge_index[1].astype(jnp.int32)
    rr = r.reshape(NW, EPW)
    rmain = rr[:, :GB_FULL * GB_ROWS].reshape(NW, GB_FULL, GB_ROWS)
    rtail = rr[:, GB_FULL * GB_ROWS:]
    zeros = jnp.zeros((STRIPE, H), f32)

    nf = _encoder(node_attr, params['node_encoder'], NB)
    ef = _encoder(edge_attr, params['edge_encoder'], EB)

    blocks = params['blocks']
    w0s = [blk['edge_mlp']['W'][0] for blk in blocks]
    a_tab, b_tab = _ab(nf, w0s[0][:H], w0s[0][H:2 * H])
    for i, blk in enumerate(blocks):
        pE, pN = blk['edge_mlp'], blk['node_mlp']
        gsum = _gather_sc(a_tab, b_tab, s, r)
        e_new = _edge_mlp(gsum, ef, pE)
        parts = _scatter_sc(e_new, rmain, rtail, zeros)
        w0_next = w0s[i + 1] if i + 1 < len(blocks) else w0s[0]
        nf, a_tab, b_tab = _node_mlp(nf, parts[0, :N], parts[1, :N], pN,
                                     w0_next)
        ef = e_new
    return _decoder(nf, params['decoder'], NB)
